# Initial kernel scaffold; baseline (speedup 1.0000x reference)
import functools

import jax
import jax.numpy as jnp
from jax import lax
from jax.experimental import pallas as pl
from jax.experimental.pallas import tpu as pltpu
from jax.experimental.pallas import tpu_sc as plsc

N = 10000
E = 320000
D = 128
NREL = 4
B = 4096
HIDDEN = 256
VOCAB = 8

NC = 2
NS = 16
NW = NC * NS
EPW = E // NW
SUB = 80
GRP = 5
BLK = SUB * GRP
NBLK = EPW // BLK
NPAD = 40960
ROWS_PER_TEC = N // NS
CNT_SL = NPAD // NS

_mesh = plsc.VectorSubcoreMesh(core_axis_name="c", subcore_axis_name="s")


def _wid():
    return lax.axis_index("s") * NC + lax.axis_index("c")


def _mm_body(x_ref, wrel_ref, wroot_ref, y_ref, root_ref):
    xb = x_ref[...]
    for r in range(NREL):
        y_ref[r] = jnp.dot(xb, wrel_ref[r], preferred_element_type=jnp.float32)
    root_ref[...] = jnp.dot(xb, wroot_ref[...], preferred_element_type=jnp.float32)


def _mm_call(x, W_rel, W_root):
    bm = 400
    grid = (N // bm,)
    return pl.pallas_call(
        _mm_body,
        grid=grid,
        in_specs=[
            pl.BlockSpec((bm, D), lambda i: (i, 0)),
            pl.BlockSpec((NREL, D, D), lambda i: (0, 0, 0)),
            pl.BlockSpec((D, D), lambda i: (0, 0)),
        ],
        out_specs=[
            pl.BlockSpec((NREL, bm, D), lambda i: (0, i, 0)),
            pl.BlockSpec((bm, D), lambda i: (i, 0)),
        ],
        out_shape=[
            jax.ShapeDtypeStruct((NREL, N, D), jnp.float32),
            jax.ShapeDtypeStruct((N, D), jnp.float32),
        ],
    )(x, W_rel, W_root)


def _cnt_body(dst_hbm, et_hbm, out_hbm, dbuf, tbuf, sidx, ones, zbuf, acc):
    c = lax.axis_index("c")
    s = lax.axis_index("s")
    wid = _wid()

    pltpu.sync_copy(dst_hbm.at[pl.ds(wid * EPW, EPW)], dbuf)
    pltpu.sync_copy(et_hbm.at[pl.ds(wid * EPW, EPW)], tbuf)
    for k in range(SUB // 16):
        ones[pl.ds(k * 16, 16)] = jnp.full((16,), 1.0, jnp.float32)

    def zb(i, _):
        zbuf[pl.ds(i * 16, 16)] = jnp.zeros((16,), jnp.float32)
        return 0
    lax.fori_loop(0, CNT_SL // 16, zb, 0)
    pltpu.sync_copy(zbuf, acc.at[pl.ds(s * CNT_SL, CNT_SL)])
    plsc.subcore_barrier()

    def body(j, _):
        base = j * SUB
        for k in range(SUB // 16):
            d = dbuf[pl.ds(base + k * 16, 16)]
            t = tbuf[pl.ds(base + k * 16, 16)]
            sidx[pl.ds(k * 16, 16)] = d * NREL + t
        pltpu.sync_copy(ones, acc.at[sidx], add=True)
        return 0
    lax.fori_loop(0, EPW // SUB, body, 0)

    plsc.subcore_barrier()
    pltpu.sync_copy(acc.at[pl.ds(s * CNT_SL, CNT_SL)],
                    out_hbm.at[c, pl.ds(s * CNT_SL, CNT_SL)])


_cnt_call = functools.partial(
    pl.kernel,
    out_type=jax.ShapeDtypeStruct((NC, NPAD), jnp.float32),
    mesh=_mesh,
    scratch_types=[
        pltpu.VMEM((EPW,), jnp.int32),
        pltpu.VMEM((EPW,), jnp.int32),
        pltpu.VMEM((SUB,), jnp.int32),
        pltpu.VMEM((SUB,), jnp.float32),
        pltpu.VMEM((CNT_SL,), jnp.float32),
        pltpu.VMEM_SHARED((NPAD,), jnp.float32),
    ],
)(_cnt_body)


def _main_body(y_hbm, src_hbm, dst_hbm, et_hbm, cnt_hbm, out_hbm,
               inv, ctmp, sbuf, dbuf, tbuf, rows, scale,
               gidx0, gidx1, gidx2, gidx3, gidx4,
               didx0, didx1, didx2, didx3, didx4, sem, acc):
    c = lax.axis_index("c")
    s = lax.axis_index("s")
    wid = _wid()
    gidx = [gidx0, gidx1, gidx2, gidx3, gidx4]
    didx = [didx0, didx1, didx2, didx3, didx4]

    pltpu.sync_copy(cnt_hbm.at[0], inv)
    for blk in range(NPAD // CNT_SL):
        pltpu.sync_copy(cnt_hbm.at[1, pl.ds(blk * CNT_SL, CNT_SL)], ctmp)

        def inv_body(k, _, blk=blk):
            off = blk * CNT_SL + k * 16
            v = inv[pl.ds(off, 16)] + ctmp[pl.ds(k * 16, 16)]
            inv[pl.ds(off, 16)] = 1.0 / jnp.maximum(v, 1.0)
            return 0
        lax.fori_loop(0, CNT_SL // 16, inv_body, 0)

    pltpu.sync_copy(src_hbm.at[pl.ds(wid * EPW, EPW)], sbuf)
    pltpu.sync_copy(dst_hbm.at[pl.ds(wid * EPW, EPW)], dbuf)
    pltpu.sync_copy(et_hbm.at[pl.ds(wid * EPW, EPW)], tbuf)

    def zrow(r, _):
        for cc in range(D // 16):
            rows[r, pl.ds(cc * 16, 16)] = jnp.zeros((16,), jnp.float32)
        return 0
    lax.fori_loop(0, BLK, zrow, 0)
    pltpu.sync_copy(rows, acc.at[pl.ds(s * ROWS_PER_TEC, BLK)])
    pltpu.sync_copy(rows.at[pl.ds(0, ROWS_PER_TEC - BLK)],
                    acc.at[pl.ds(s * ROWS_PER_TEC + BLK, ROWS_PER_TEC - BLK)])
    plsc.subcore_barrier()

    def block(j, _):
        base = j * BLK
        for k in range(GRP):
            for m in range(SUB // 16):
                off = base + k * SUB + m * 16
                sv = sbuf[pl.ds(off, 16)]
                dv = dbuf[pl.ds(off, 16)]
                tv = tbuf[pl.ds(off, 16)]
                gidx[k][pl.ds(m * 16, 16)] = tv * N + sv
                didx[k][pl.ds(m * 16, 16)] = dv
        cps = [pltpu.async_copy(y_hbm.at[gidx[k]],
                                rows.at[pl.ds(k * SUB, SUB)], sem)
               for k in range(GRP)]
        for k in range(GRP):
            for m in range(SUB // 16):
                off = base + k * SUB + m * 16
                dv = dbuf[pl.ds(off, 16)]
                tv = tbuf[pl.ds(off, 16)]
                sc16 = plsc.load_gather(inv, [dv * NREL + tv])
                scale[pl.ds(k * SUB + m * 16, 16)] = sc16
        for cp in cps:
            cp.wait()

        def mrow(r, _):
            sc = scale[r]
            for cc in range(D // 16):
                rows[r, pl.ds(cc * 16, 16)] = rows[r, pl.ds(cc * 16, 16)] * sc
            return 0
        lax.fori_loop(0, BLK, mrow, 0)
        for k in range(GRP):
            pltpu.sync_copy(rows.at[pl.ds(k * SUB, SUB)],
                            acc.at[didx[k]], add=True)
        return 0
    lax.fori_loop(0, NBLK, block, 0)

    plsc.subcore_barrier()
    r0 = s * ROWS_PER_TEC
    pltpu.sync_copy(acc.at[pl.ds(r0, BLK)], out_hbm.at[c, pl.ds(r0, BLK)])
    pltpu.sync_copy(acc.at[pl.ds(r0 + BLK, ROWS_PER_TEC - BLK)],
                    out_hbm.at[c, pl.ds(r0 + BLK, ROWS_PER_TEC - BLK)])


_main_call = functools.partial(
    pl.kernel,
    out_type=jax.ShapeDtypeStruct((NC, N, D), jnp.float32),
    mesh=_mesh,
    scratch_types=[
        pltpu.VMEM((NPAD,), jnp.float32),
        pltpu.VMEM((CNT_SL,), jnp.float32),
        pltpu.VMEM((EPW,), jnp.int32),
        pltpu.VMEM((EPW,), jnp.int32),
        pltpu.VMEM((EPW,), jnp.int32),
        pltpu.VMEM((BLK, D), jnp.float32),
        pltpu.VMEM((BLK,), jnp.float32),
        pltpu.VMEM((SUB,), jnp.int32),
        pltpu.VMEM((SUB,), jnp.int32),
        pltpu.VMEM((SUB,), jnp.int32),
        pltpu.VMEM((SUB,), jnp.int32),
        pltpu.VMEM((SUB,), jnp.int32),
        pltpu.VMEM((SUB,), jnp.int32),
        pltpu.VMEM((SUB,), jnp.int32),
        pltpu.VMEM((SUB,), jnp.int32),
        pltpu.VMEM((SUB,), jnp.int32),
        pltpu.VMEM((SUB,), jnp.int32),
        pltpu.SemaphoreType.DMA,
        pltpu.VMEM_SHARED((N, D), jnp.float32),
    ],
)(_main_body)


_EPT = B // NW


def _emb_body(root_hbm, m0_hbm, m1_hbm, b_hbm, nest_hbm, food_hbm,
              ne_hbm, fe_hbm, idxb, rbuf, m0b, m1b, bbuf):
    wid = _wid()
    pltpu.sync_copy(b_hbm, bbuf)
    base = wid * _EPT
    for idx_hbm, o_hbm in ((nest_hbm, ne_hbm), (food_hbm, fe_hbm)):
        pltpu.sync_copy(idx_hbm.at[pl.ds(base, _EPT)], idxb)
        pltpu.sync_copy(root_hbm.at[idxb], rbuf)
        pltpu.sync_copy(m0_hbm.at[idxb], m0b)
        pltpu.sync_copy(m1_hbm.at[idxb], m1b)

        def row(r, _):
            for cc in range(D // 16):
                ds = pl.ds(cc * 16, 16)
                v = rbuf[r, ds] + m0b[r, ds] + m1b[r, ds] + bbuf[ds]
                rbuf[r, ds] = jnp.maximum(v, 0.0)
            return 0
        lax.fori_loop(0, _EPT, row, 0)
        pltpu.sync_copy(rbuf, o_hbm.at[pl.ds(base, _EPT)])


_emb_call = functools.partial(
    pl.kernel,
    out_type=[jax.ShapeDtypeStruct((B, D), jnp.float32),
              jax.ShapeDtypeStruct((B, D), jnp.float32)],
    mesh=_mesh,
    scratch_types=[
        pltpu.VMEM((_EPT,), jnp.int32),
        pltpu.VMEM((_EPT, D), jnp.float32),
        pltpu.VMEM((_EPT, D), jnp.float32),
        pltpu.VMEM((_EPT, D), jnp.float32),
        pltpu.VMEM((D,), jnp.float32),
    ],
)(_emb_body)


def _head_body(ne_ref, fe_ref, wfc_ref, bfc_ref, wdir_ref, bdir_ref,
               wdist_ref, bdist_ref, la_ref, tb_ref):
    hid = jnp.dot(ne_ref[...], wfc_ref[:D], preferred_element_type=jnp.float32)
    hid = hid + jnp.dot(fe_ref[...], wfc_ref[D:],
                        preferred_element_type=jnp.float32)
    hid = jnp.maximum(hid + bfc_ref[...], 0.0)
    logit = jnp.dot(hid, wdir_ref[...], preferred_element_type=jnp.float32)
    logit = logit + bdir_ref[...]
    m = jnp.max(logit, axis=-1, keepdims=True)
    lse = jnp.log(jnp.sum(jnp.exp(logit - m), axis=-1, keepdims=True)) + m
    la_ref[...] = logit - lse
    tb_ref[...] = (jnp.dot(hid, wdist_ref[...],
                           preferred_element_type=jnp.float32)
                   + bdist_ref[...])


def _head_call(ne, fe, W_fc, b_fc, W_dir, b_dir, W_dist, b_dist):
    bm = 512
    grid = (B // bm,)
    return pl.pallas_call(
        _head_body,
        grid=grid,
        in_specs=[
            pl.BlockSpec((bm, D), lambda i: (i, 0)),
            pl.BlockSpec((bm, D), lambda i: (i, 0)),
            pl.BlockSpec((2 * D, HIDDEN), lambda i: (0, 0)),
            pl.BlockSpec((1, HIDDEN), lambda i: (0, 0)),
            pl.BlockSpec((HIDDEN, VOCAB), lambda i: (0, 0)),
            pl.BlockSpec((1, VOCAB), lambda i: (0, 0)),
            pl.BlockSpec((HIDDEN, 1), lambda i: (0, 0)),
            pl.BlockSpec((1, 1), lambda i: (0, 0)),
        ],
        out_specs=[
            pl.BlockSpec((bm, VOCAB), lambda i: (i, 0)),
            pl.BlockSpec((bm, 1), lambda i: (i, 0)),
        ],
        out_shape=[
            jax.ShapeDtypeStruct((B, VOCAB), jnp.float32),
            jax.ShapeDtypeStruct((B, 1), jnp.float32),
        ],
    )(ne, fe, W_fc, b_fc, W_dir, b_dir, W_dist, b_dist)


def kernel(x, edge_index, edge_type, nest, food, W_rel, W_root, b_rgcn,
           W_fc, b_fc, W_dir, b_dir, W_dist, b_dist):
    src = edge_index[0].astype(jnp.int32)
    dst = edge_index[1].astype(jnp.int32)
    et = edge_type.astype(jnp.int32)
    nest32 = nest.astype(jnp.int32)
    food32 = food.astype(jnp.int32)

    Y, root = _mm_call(x, W_rel, W_root)
    Y2 = Y.reshape(NREL * N, D)
    cnt2 = _cnt_call(dst, et)
    msgp = _main_call(Y2, src, dst, et, cnt2)
    ne, fe = _emb_call(root, msgp[0], msgp[1], b_rgcn, nest32, food32)
    la, tb = _head_call(ne, fe, W_fc, b_fc.reshape(1, -1),
                        W_dir, b_dir.reshape(1, -1),
                        W_dist, b_dist.reshape(1, -1))
    return (la, tb)


# speedup vs baseline: 13.6319x; 13.6319x over previous
import functools

import jax
import jax.numpy as jnp
from jax import lax
from jax.experimental import pallas as pl
from jax.experimental.pallas import tpu as pltpu
from jax.experimental.pallas import tpu_sc as plsc

N = 10000
E = 320000
D = 128
NREL = 4
B = 4096
HIDDEN = 256
VOCAB = 8

NC = 2
NS = 16
NW = NC * NS
EPW = E // NW
SUB = 80
GRP = 5
BLK = SUB * GRP
NBLK = EPW // BLK
NPAD = 40960
ZROWS = 624
CNT_SL = NPAD // NS

_mesh = plsc.VectorSubcoreMesh(core_axis_name="c", subcore_axis_name="s")


def _wid():
    return lax.axis_index("s") * NC + lax.axis_index("c")


def _mm_body(x_ref, wrel_ref, wroot_ref, y_ref, root_ref):
    xb = x_ref[...]
    for r in range(NREL):
        y_ref[r] = jnp.dot(xb, wrel_ref[r], preferred_element_type=jnp.float32)
    root_ref[...] = jnp.dot(xb, wroot_ref[...], preferred_element_type=jnp.float32)


def _mm_call(x, W_rel, W_root):
    bm = 400
    grid = (N // bm,)
    return pl.pallas_call(
        _mm_body,
        grid=grid,
        in_specs=[
            pl.BlockSpec((bm, D), lambda i: (i, 0)),
            pl.BlockSpec((NREL, D, D), lambda i: (0, 0, 0)),
            pl.BlockSpec((D, D), lambda i: (0, 0)),
        ],
        out_specs=[
            pl.BlockSpec((NREL, bm, D), lambda i: (0, i, 0)),
            pl.BlockSpec((bm, D), lambda i: (i, 0)),
        ],
        out_shape=[
            jax.ShapeDtypeStruct((NREL, N, D), jnp.float32),
            jax.ShapeDtypeStruct((N, D), jnp.float32),
        ],
    )(x, W_rel, W_root)


SUBC = 128
NFULL = EPW // SUBC
TAIL = EPW - NFULL * SUBC
MASK14 = 16383


def _unpack(p):
    sv = p & MASK14
    dv = (p >> 14) & MASK14
    tv = p >> 28
    return sv, dv, tv


def _cnt_body(ep_hbm, out_hbm, pbuf, sidx, sidxt, ones, zbuf, acc):
    c = lax.axis_index("c")
    s = lax.axis_index("s")
    wid = _wid()

    pltpu.sync_copy(ep_hbm.at[pl.ds(wid * EPW, EPW)], pbuf)
    for k in range(SUBC // 16):
        ones[pl.ds(k * 16, 16)] = jnp.full((16,), 1.0, jnp.float32)

    def zb(i, _):
        zbuf[pl.ds(i * 16, 16)] = jnp.zeros((16,), jnp.float32)
        return 0
    lax.fori_loop(0, CNT_SL // 16, zb, 0)
    pltpu.sync_copy(zbuf, acc.at[pl.ds(s * CNT_SL, CNT_SL)])
    plsc.subcore_barrier()

    def body(j, _):
        base = j * SUBC
        for m in range(SUBC // 16):
            p = pbuf[pl.ds(base + m * 16, 16)]
            _, dv, tv = _unpack(p)
            sidx[pl.ds(m * 16, 16)] = dv * NREL + tv
        pltpu.sync_copy(ones, acc.at[sidx], add=True)
        return 0
    lax.fori_loop(0, NFULL, body, 0)
    p = pbuf[pl.ds(NFULL * SUBC, TAIL)]
    _, dv, tv = _unpack(p)
    sidxt[...] = dv * NREL + tv
    pltpu.sync_copy(ones.at[pl.ds(0, TAIL)], acc.at[sidxt], add=True)

    plsc.subcore_barrier()
    pltpu.sync_copy(acc.at[pl.ds(s * CNT_SL, CNT_SL)],
                    out_hbm.at[c, pl.ds(s * CNT_SL, CNT_SL)])


_cnt_call = functools.partial(
    pl.kernel,
    out_type=jax.ShapeDtypeStruct((NC, NPAD), jnp.float32),
    mesh=_mesh,
    compiler_params=pltpu.CompilerParams(needs_layout_passes=False),
    scratch_types=[
        pltpu.VMEM((EPW,), jnp.int32),
        pltpu.VMEM((SUBC,), jnp.int32),
        pltpu.VMEM((TAIL,), jnp.int32),
        pltpu.VMEM((SUBC,), jnp.float32),
        pltpu.VMEM((CNT_SL,), jnp.float32),
        pltpu.VMEM_SHARED((NPAD,), jnp.float32),
    ],
)(_cnt_body)


def _inv_body(cnt_ref, inv_ref):
    cb = cnt_ref[...]
    inv_ref[...] = 1.0 / jnp.maximum(cb[0] + cb[1], 1.0)


def _inv_call(cnt3):
    nr = NPAD // D
    return pl.pallas_call(
        _inv_body,
        grid=(1,),
        in_specs=[pl.BlockSpec((NC, nr, D), lambda i: (0, 0, 0))],
        out_specs=pl.BlockSpec((nr, D), lambda i: (0, 0)),
        out_shape=jax.ShapeDtypeStruct((nr, D), jnp.float32),
    )(cnt3)


def _main_body(y_hbm, ep_hbm, inv_hbm, out_hbm,
               pbuf, rows, scale, gidx, didx, sidx,
               gidxt, didxt, sidxt, sem, acc):
    c = lax.axis_index("c")
    s = lax.axis_index("s")
    wid = _wid()

    pltpu.sync_copy(ep_hbm.at[pl.ds(wid * EPW, EPW)], pbuf)

    def zrow(r, _):
        for cc in range(D // 16):
            rows[r, pl.ds(cc * 16, 16)] = jnp.zeros((16,), jnp.float32)
        return 0
    lax.fori_loop(0, SUBC, zrow, 0)
    z0 = s * ZROWS
    for q in range(ZROWS // SUBC):
        pltpu.sync_copy(rows, acc.at[pl.ds(z0 + q * SUBC, SUBC)])
    rem = ZROWS - (ZROWS // SUBC) * SUBC
    pltpu.sync_copy(rows.at[pl.ds(0, rem)],
                    acc.at[pl.ds(z0 + (ZROWS // SUBC) * SUBC, rem)])

    @pl.when(s == NS - 1)
    def _():
        pltpu.sync_copy(rows.at[pl.ds(0, N - NS * ZROWS)],
                        acc.at[pl.ds(NS * ZROWS, N - NS * ZROWS)])
    plsc.subcore_barrier()

    def block(j, _):
        base = j * SUBC
        for m in range(SUBC // 16):
            p = pbuf[pl.ds(base + m * 16, 16)]
            sv, dv, tv = _unpack(p)
            gidx[pl.ds(m * 16, 16)] = tv * N + sv
            didx[pl.ds(m * 16, 16)] = dv
            sidx[pl.ds(m * 16, 16)] = dv * NREL + tv
        cp1 = pltpu.async_copy(y_hbm.at[gidx], rows, sem)
        cp2 = pltpu.async_copy(inv_hbm.at[sidx], scale, sem)
        cp1.wait()
        cp2.wait()

        def mrow(r, _):
            sc = plsc.load_gather(scale, [jnp.full((16,), r, jnp.int32)])
            for cc in range(D // 16):
                rows[r, pl.ds(cc * 16, 16)] = rows[r, pl.ds(cc * 16, 16)] * sc
            return 0
        lax.fori_loop(0, SUBC, mrow, 0)
        pltpu.sync_copy(rows, acc.at[didx], add=True)
        return 0
    lax.fori_loop(0, NFULL, block, 0)

    p = pbuf[pl.ds(NFULL * SUBC, TAIL)]
    sv, dv, tv = _unpack(p)
    gidxt[...] = tv * N + sv
    didxt[...] = dv
    sidxt[...] = dv * NREL + tv
    cp1 = pltpu.async_copy(y_hbm.at[gidxt], rows.at[pl.ds(0, TAIL)], sem)
    cp2 = pltpu.async_copy(inv_hbm.at[sidxt], scale.at[pl.ds(0, TAIL)], sem)
    cp1.wait()
    cp2.wait()

    def mrowt(r, _):
        sc = plsc.load_gather(scale, [jnp.full((16,), r, jnp.int32)])
        for cc in range(D // 16):
            rows[r, pl.ds(cc * 16, 16)] = rows[r, pl.ds(cc * 16, 16)] * sc
        return 0
    lax.fori_loop(0, TAIL, mrowt, 0)
    pltpu.sync_copy(rows.at[pl.ds(0, TAIL)], acc.at[didxt], add=True)

    plsc.subcore_barrier()
    r0 = s * ZROWS
    for q in range(ZROWS // SUBC):
        pltpu.sync_copy(acc.at[pl.ds(r0 + q * SUBC, SUBC)],
                        out_hbm.at[c, pl.ds(r0 + q * SUBC, SUBC)])
    pltpu.sync_copy(acc.at[pl.ds(r0 + (ZROWS // SUBC) * SUBC, rem)],
                    out_hbm.at[c, pl.ds(r0 + (ZROWS // SUBC) * SUBC, rem)])

    @pl.when(s == NS - 1)
    def _():
        pltpu.sync_copy(acc.at[pl.ds(NS * ZROWS, N - NS * ZROWS)],
                        out_hbm.at[c, pl.ds(NS * ZROWS, N - NS * ZROWS)])


_main_call = functools.partial(
    pl.kernel,
    out_type=jax.ShapeDtypeStruct((NC, N, D), jnp.float32),
    mesh=_mesh,
    compiler_params=pltpu.CompilerParams(needs_layout_passes=False),
    scratch_types=[
        pltpu.VMEM((EPW,), jnp.int32),
        pltpu.VMEM((SUBC, D), jnp.float32),
        pltpu.VMEM((SUBC,), jnp.float32),
        pltpu.VMEM((SUBC,), jnp.int32),
        pltpu.VMEM((SUBC,), jnp.int32),
        pltpu.VMEM((SUBC,), jnp.int32),
        pltpu.VMEM((TAIL,), jnp.int32),
        pltpu.VMEM((TAIL,), jnp.int32),
        pltpu.VMEM((TAIL,), jnp.int32),
        pltpu.SemaphoreType.DMA,
        pltpu.VMEM_SHARED((N, D), jnp.float32),
    ],
)(_main_body)


_EPT = B // NW


def _emb_body(root_hbm, m0_hbm, m1_hbm, b_hbm, nest_hbm, food_hbm,
              ne_hbm, fe_hbm, idxb, rbuf, m0b, m1b, bbuf):
    wid = _wid()
    pltpu.sync_copy(b_hbm, bbuf)
    base = wid * _EPT
    for idx_hbm, o_hbm in ((nest_hbm, ne_hbm), (food_hbm, fe_hbm)):
        pltpu.sync_copy(idx_hbm.at[pl.ds(base, _EPT)], idxb)
        pltpu.sync_copy(root_hbm.at[idxb], rbuf)
        pltpu.sync_copy(m0_hbm.at[idxb], m0b)
        pltpu.sync_copy(m1_hbm.at[idxb], m1b)

        def row(r, _):
            for cc in range(D // 16):
                ds = pl.ds(cc * 16, 16)
                v = rbuf[r, ds] + m0b[r, ds] + m1b[r, ds] + bbuf[ds]
                rbuf[r, ds] = jnp.maximum(v, 0.0)
            return 0
        lax.fori_loop(0, _EPT, row, 0)
        pltpu.sync_copy(rbuf, o_hbm.at[pl.ds(base, _EPT)])


_emb_call = functools.partial(
    pl.kernel,
    out_type=[jax.ShapeDtypeStruct((B, D), jnp.float32),
              jax.ShapeDtypeStruct((B, D), jnp.float32)],
    mesh=_mesh,
    compiler_params=pltpu.CompilerParams(needs_layout_passes=False),
    scratch_types=[
        pltpu.VMEM((_EPT,), jnp.int32),
        pltpu.VMEM((_EPT, D), jnp.float32),
        pltpu.VMEM((_EPT, D), jnp.float32),
        pltpu.VMEM((_EPT, D), jnp.float32),
        pltpu.VMEM((D,), jnp.float32),
    ],
)(_emb_body)


def _head_body(ne_ref, fe_ref, wfc_ref, bfc_ref, wdir_ref, bdir_ref,
               wdist_ref, bdist_ref, la_ref, tb_ref):
    hid = jnp.dot(ne_ref[...], wfc_ref[:D], preferred_element_type=jnp.float32)
    hid = hid + jnp.dot(fe_ref[...], wfc_ref[D:],
                        preferred_element_type=jnp.float32)
    hid = jnp.maximum(hid + bfc_ref[...], 0.0)
    logit = jnp.dot(hid, wdir_ref[...], preferred_element_type=jnp.float32)
    logit = logit + bdir_ref[...]
    m = jnp.max(logit, axis=-1, keepdims=True)
    lse = jnp.log(jnp.sum(jnp.exp(logit - m), axis=-1, keepdims=True)) + m
    la_ref[...] = logit - lse
    tb_ref[...] = (jnp.dot(hid, wdist_ref[...],
                           preferred_element_type=jnp.float32)
                   + bdist_ref[...])


def _head_call(ne, fe, W_fc, b_fc, W_dir, b_dir, W_dist, b_dist):
    bm = 512
    grid = (B // bm,)
    return pl.pallas_call(
        _head_body,
        grid=grid,
        in_specs=[
            pl.BlockSpec((bm, D), lambda i: (i, 0)),
            pl.BlockSpec((bm, D), lambda i: (i, 0)),
            pl.BlockSpec((2 * D, HIDDEN), lambda i: (0, 0)),
            pl.BlockSpec((1, HIDDEN), lambda i: (0, 0)),
            pl.BlockSpec((HIDDEN, VOCAB), lambda i: (0, 0)),
            pl.BlockSpec((1, VOCAB), lambda i: (0, 0)),
            pl.BlockSpec((HIDDEN, 1), lambda i: (0, 0)),
            pl.BlockSpec((1, 1), lambda i: (0, 0)),
        ],
        out_specs=[
            pl.BlockSpec((bm, VOCAB), lambda i: (i, 0)),
            pl.BlockSpec((bm, 1), lambda i: (i, 0)),
        ],
        out_shape=[
            jax.ShapeDtypeStruct((B, VOCAB), jnp.float32),
            jax.ShapeDtypeStruct((B, 1), jnp.float32),
        ],
    )(ne, fe, W_fc, b_fc, W_dir, b_dir, W_dist, b_dist)


def kernel(x, edge_index, edge_type, nest, food, W_rel, W_root, b_rgcn,
           W_fc, b_fc, W_dir, b_dir, W_dist, b_dist):
    src = edge_index[0].astype(jnp.int32)
    dst = edge_index[1].astype(jnp.int32)
    et = edge_type.astype(jnp.int32)
    nest32 = nest.astype(jnp.int32)
    food32 = food.astype(jnp.int32)

    epack = src + dst * 16384 + et * 268435456

    Y, root = _mm_call(x, W_rel, W_root)
    Y2 = Y.reshape(NREL * N, D)
    cnt2 = _cnt_call(epack)
    inv = _inv_call(cnt2.reshape(NC, NPAD // D, D)).reshape(NPAD)
    msgp = _main_call(Y2, epack, inv)
    ne, fe = _emb_call(root, msgp[0], msgp[1], b_rgcn, nest32, food32)
    la, tb = _head_call(ne, fe, W_fc, b_fc.reshape(1, -1),
                        W_dir, b_dir.reshape(1, -1),
                        W_dist, b_dist.reshape(1, -1))
    return (la, tb)


# speedup vs baseline: 18.5392x; 1.3600x over previous
import functools

import jax
import jax.numpy as jnp
from jax import lax
from jax.experimental import pallas as pl
from jax.experimental.pallas import tpu as pltpu
from jax.experimental.pallas import tpu_sc as plsc

N = 10000
E = 320000
D = 128
NREL = 4
B = 4096
HIDDEN = 256
VOCAB = 8

NC = 2
NS = 16
NW = NC * NS
EPW = E // NW
SUB = 80
GRP = 5
BLK = SUB * GRP
NBLK = EPW // BLK
NPAD = 40960
ZROWS = 624
CNT_SL = NPAD // NS

_mesh = plsc.VectorSubcoreMesh(core_axis_name="c", subcore_axis_name="s")


def _wid():
    return lax.axis_index("s") * NC + lax.axis_index("c")


def _mm_body(x_ref, wrel_ref, wroot_ref, y_ref, root_ref):
    xb = x_ref[...]
    for r in range(NREL):
        y_ref[r] = jnp.dot(xb, wrel_ref[r], preferred_element_type=jnp.float32)
    root_ref[...] = jnp.dot(xb, wroot_ref[...], preferred_element_type=jnp.float32)


def _mm_call(x, W_rel, W_root):
    bm = 400
    grid = (N // bm,)
    return pl.pallas_call(
        _mm_body,
        grid=grid,
        in_specs=[
            pl.BlockSpec((bm, D), lambda i: (i, 0)),
            pl.BlockSpec((NREL, D, D), lambda i: (0, 0, 0)),
            pl.BlockSpec((D, D), lambda i: (0, 0)),
        ],
        out_specs=[
            pl.BlockSpec((NREL, bm, D), lambda i: (0, i, 0)),
            pl.BlockSpec((bm, D), lambda i: (i, 0)),
        ],
        out_shape=[
            jax.ShapeDtypeStruct((NREL, N, D), jnp.float32),
            jax.ShapeDtypeStruct((N, D), jnp.float32),
        ],
    )(x, W_rel, W_root)


SUBC = 128
NFULL = EPW // SUBC
TAIL = EPW - NFULL * SUBC
MASK14 = 16383


def _unpack(p):
    sv = p & MASK14
    dv = (p >> 14) & MASK14
    tv = p >> 28
    return sv, dv, tv


def _cnt_body(ep_hbm, out_hbm, pbuf, sidx, sidxt, ones, zbuf, acc):
    c = lax.axis_index("c")
    s = lax.axis_index("s")
    wid = _wid()

    pltpu.sync_copy(ep_hbm.at[pl.ds(wid * EPW, EPW)], pbuf)
    for k in range(SUBC // 16):
        ones[pl.ds(k * 16, 16)] = jnp.full((16,), 1.0, jnp.float32)

    def zb(i, _):
        zbuf[pl.ds(i * 16, 16)] = jnp.zeros((16,), jnp.float32)
        return 0
    lax.fori_loop(0, CNT_SL // 16, zb, 0)
    pltpu.sync_copy(zbuf, acc.at[pl.ds(s * CNT_SL, CNT_SL)])
    plsc.subcore_barrier()

    def body(j, _):
        base = j * SUBC
        for m in range(SUBC // 16):
            p = pbuf[pl.ds(base + m * 16, 16)]
            _, dv, tv = _unpack(p)
            sidx[pl.ds(m * 16, 16)] = dv * NREL + tv
        pltpu.sync_copy(ones, acc.at[sidx], add=True)
        return 0
    lax.fori_loop(0, NFULL, body, 0)
    p = pbuf[pl.ds(NFULL * SUBC, TAIL)]
    _, dv, tv = _unpack(p)
    sidxt[...] = dv * NREL + tv
    pltpu.sync_copy(ones.at[pl.ds(0, TAIL)], acc.at[sidxt], add=True)

    plsc.subcore_barrier()
    pltpu.sync_copy(acc.at[pl.ds(s * CNT_SL, CNT_SL)],
                    out_hbm.at[c, pl.ds(s * CNT_SL, CNT_SL)])


_cnt_call = functools.partial(
    pl.kernel,
    out_type=jax.ShapeDtypeStruct((NC, NPAD), jnp.float32),
    mesh=_mesh,
    compiler_params=pltpu.CompilerParams(needs_layout_passes=False),
    scratch_types=[
        pltpu.VMEM((EPW,), jnp.int32),
        pltpu.VMEM((SUBC,), jnp.int32),
        pltpu.VMEM((TAIL,), jnp.int32),
        pltpu.VMEM((SUBC,), jnp.float32),
        pltpu.VMEM((CNT_SL,), jnp.float32),
        pltpu.VMEM_SHARED((NPAD,), jnp.float32),
    ],
)(_cnt_body)


def _inv_body(cnt_ref, inv_ref):
    cb = cnt_ref[...]
    inv_ref[...] = 1.0 / jnp.maximum(cb[0] + cb[1], 1.0)


def _inv_call(cnt3):
    nr = NPAD // D
    return pl.pallas_call(
        _inv_body,
        grid=(1,),
        in_specs=[pl.BlockSpec((NC, nr, D), lambda i: (0, 0, 0))],
        out_specs=pl.BlockSpec((nr, D), lambda i: (0, 0)),
        out_shape=jax.ShapeDtypeStruct((nr, D), jnp.float32),
    )(cnt3)


NITER = NFULL // 2


def _main_body(y_hbm, ep_hbm, inv_hbm, out_hbm,
               pbuf, rows0, rows1, scale0, scale1,
               gidx0, didx0, sidx0, gidx1, didx1, sidx1,
               gidxt, didxt, sidxt,
               semg0, semg1, sems0, sems1, acc):
    c = lax.axis_index("c")
    s = lax.axis_index("s")
    wid = _wid()

    pltpu.sync_copy(ep_hbm.at[pl.ds(wid * EPW, EPW)], pbuf)

    def zrow(r, _):
        for cc in range(D // 16):
            rows0[r, pl.ds(cc * 16, 16)] = jnp.zeros((16,), jnp.float32)
        return 0
    lax.fori_loop(0, SUBC, zrow, 0)
    z0 = s * ZROWS
    for q in range(ZROWS // SUBC):
        pltpu.sync_copy(rows0, acc.at[pl.ds(z0 + q * SUBC, SUBC)])
    rem = ZROWS - (ZROWS // SUBC) * SUBC
    pltpu.sync_copy(rows0.at[pl.ds(0, rem)],
                    acc.at[pl.ds(z0 + (ZROWS // SUBC) * SUBC, rem)])

    @pl.when(s == NS - 1)
    def _():
        pltpu.sync_copy(rows0.at[pl.ds(0, N - NS * ZROWS)],
                        acc.at[pl.ds(NS * ZROWS, N - NS * ZROWS)])
    plsc.subcore_barrier()

    def prep(j, gi, di, si, rw, sc, sg):
        base = j * SUBC
        for m in range(SUBC // 16):
            p = pbuf[pl.ds(base + m * 16, 16)]
            sv, dv, tv = _unpack(p)
            gi[pl.ds(m * 16, 16)] = tv * N + sv
            di[pl.ds(m * 16, 16)] = dv
            si[pl.ds(m * 16, 16)] = dv * NREL + tv
        pltpu.async_copy(y_hbm.at[gi], rw, sg)
        pltpu.async_copy(inv_hbm.at[si], sc, sg)

    def proc(gi, di, si, rw, sc, sg, ss):
        pltpu.make_async_copy(y_hbm.at[gi], rw, sg).wait()
        pltpu.make_async_copy(inv_hbm.at[si], sc, sg).wait()

        def mrow(r, _):
            s16 = plsc.load_gather(sc, [jnp.full((16,), r, jnp.int32)])
            for cc in range(D // 16):
                rw[r, pl.ds(cc * 16, 16)] = rw[r, pl.ds(cc * 16, 16)] * s16
            return 0
        lax.fori_loop(0, SUBC, mrow, 0)
        pltpu.async_copy(rw, acc.at[di], ss, add=True)

    def drain(rw, di, ss):
        pltpu.make_async_copy(rw, acc.at[di], ss).wait()

    set0 = (gidx0, didx0, sidx0, rows0, scale0, semg0)
    set1 = (gidx1, didx1, sidx1, rows1, scale1, semg1)

    prep(0, *set0)
    prep(1, *set1)

    def body(i, _):
        j0 = 2 * i
        proc(*set0[:5], semg0, sems0)
        proc(*set1[:5], semg1, sems1)

        @pl.when(i < NITER - 1)
        def _():
            drain(rows0, didx0, sems0)
            prep(j0 + 2, *set0)
            drain(rows1, didx1, sems1)
            prep(j0 + 3, *set1)
        return 0
    lax.fori_loop(0, NITER, body, 0)
    drain(rows0, didx0, sems0)
    drain(rows1, didx1, sems1)

    p = pbuf[pl.ds(NFULL * SUBC, TAIL)]
    sv, dv, tv = _unpack(p)
    gidxt[...] = tv * N + sv
    didxt[...] = dv
    sidxt[...] = dv * NREL + tv
    cp1 = pltpu.async_copy(y_hbm.at[gidxt], rows0.at[pl.ds(0, TAIL)], semg0)
    cp2 = pltpu.async_copy(inv_hbm.at[sidxt], scale0.at[pl.ds(0, TAIL)], semg0)
    cp1.wait()
    cp2.wait()

    def mrowt(r, _):
        s16 = plsc.load_gather(scale0, [jnp.full((16,), r, jnp.int32)])
        for cc in range(D // 16):
            rows0[r, pl.ds(cc * 16, 16)] = rows0[r, pl.ds(cc * 16, 16)] * s16
        return 0
    lax.fori_loop(0, TAIL, mrowt, 0)
    pltpu.sync_copy(rows0.at[pl.ds(0, TAIL)], acc.at[didxt], add=True)

    plsc.subcore_barrier()
    r0 = s * ZROWS
    for q in range(ZROWS // SUBC):
        pltpu.sync_copy(acc.at[pl.ds(r0 + q * SUBC, SUBC)],
                        out_hbm.at[c, pl.ds(r0 + q * SUBC, SUBC)])
    pltpu.sync_copy(acc.at[pl.ds(r0 + (ZROWS // SUBC) * SUBC, rem)],
                    out_hbm.at[c, pl.ds(r0 + (ZROWS // SUBC) * SUBC, rem)])

    @pl.when(s == NS - 1)
    def _():
        pltpu.sync_copy(acc.at[pl.ds(NS * ZROWS, N - NS * ZROWS)],
                        out_hbm.at[c, pl.ds(NS * ZROWS, N - NS * ZROWS)])


_main_call = functools.partial(
    pl.kernel,
    out_type=jax.ShapeDtypeStruct((NC, N, D), jnp.float32),
    mesh=_mesh,
    compiler_params=pltpu.CompilerParams(needs_layout_passes=False),
    scratch_types=[
        pltpu.VMEM((EPW,), jnp.int32),
        pltpu.VMEM((SUBC, D), jnp.float32),
        pltpu.VMEM((SUBC, D), jnp.float32),
        pltpu.VMEM((SUBC,), jnp.float32),
        pltpu.VMEM((SUBC,), jnp.float32),
        pltpu.VMEM((SUBC,), jnp.int32),
        pltpu.VMEM((SUBC,), jnp.int32),
        pltpu.VMEM((SUBC,), jnp.int32),
        pltpu.VMEM((SUBC,), jnp.int32),
        pltpu.VMEM((SUBC,), jnp.int32),
        pltpu.VMEM((SUBC,), jnp.int32),
        pltpu.VMEM((TAIL,), jnp.int32),
        pltpu.VMEM((TAIL,), jnp.int32),
        pltpu.VMEM((TAIL,), jnp.int32),
        pltpu.SemaphoreType.DMA,
        pltpu.SemaphoreType.DMA,
        pltpu.SemaphoreType.DMA,
        pltpu.SemaphoreType.DMA,
        pltpu.VMEM_SHARED((N, D), jnp.float32),
    ],
)(_main_body)


_EPT = B // NW


def _emb_body(root_hbm, m0_hbm, m1_hbm, b_hbm, nest_hbm, food_hbm,
              ne_hbm, fe_hbm, idxb, rbuf, m0b, m1b, bbuf):
    wid = _wid()
    pltpu.sync_copy(b_hbm, bbuf)
    base = wid * _EPT
    for idx_hbm, o_hbm in ((nest_hbm, ne_hbm), (food_hbm, fe_hbm)):
        pltpu.sync_copy(idx_hbm.at[pl.ds(base, _EPT)], idxb)
        pltpu.sync_copy(root_hbm.at[idxb], rbuf)
        pltpu.sync_copy(m0_hbm.at[idxb], m0b)
        pltpu.sync_copy(m1_hbm.at[idxb], m1b)

        def row(r, _):
            for cc in range(D // 16):
                ds = pl.ds(cc * 16, 16)
                v = rbuf[r, ds] + m0b[r, ds] + m1b[r, ds] + bbuf[ds]
                rbuf[r, ds] = jnp.maximum(v, 0.0)
            return 0
        lax.fori_loop(0, _EPT, row, 0)
        pltpu.sync_copy(rbuf, o_hbm.at[pl.ds(base, _EPT)])


_emb_call = functools.partial(
    pl.kernel,
    out_type=[jax.ShapeDtypeStruct((B, D), jnp.float32),
              jax.ShapeDtypeStruct((B, D), jnp.float32)],
    mesh=_mesh,
    compiler_params=pltpu.CompilerParams(needs_layout_passes=False),
    scratch_types=[
        pltpu.VMEM((_EPT,), jnp.int32),
        pltpu.VMEM((_EPT, D), jnp.float32),
        pltpu.VMEM((_EPT, D), jnp.float32),
        pltpu.VMEM((_EPT, D), jnp.float32),
        pltpu.VMEM((D,), jnp.float32),
    ],
)(_emb_body)


def _head_body(ne_ref, fe_ref, wfc_ref, bfc_ref, wdir_ref, bdir_ref,
               wdist_ref, bdist_ref, la_ref, tb_ref):
    hid = jnp.dot(ne_ref[...], wfc_ref[:D], preferred_element_type=jnp.float32)
    hid = hid + jnp.dot(fe_ref[...], wfc_ref[D:],
                        preferred_element_type=jnp.float32)
    hid = jnp.maximum(hid + bfc_ref[...], 0.0)
    logit = jnp.dot(hid, wdir_ref[...], preferred_element_type=jnp.float32)
    logit = logit + bdir_ref[...]
    m = jnp.max(logit, axis=-1, keepdims=True)
    lse = jnp.log(jnp.sum(jnp.exp(logit - m), axis=-1, keepdims=True)) + m
    la_ref[...] = logit - lse
    tb_ref[...] = (jnp.dot(hid, wdist_ref[...],
                           preferred_element_type=jnp.float32)
                   + bdist_ref[...])


def _head_call(ne, fe, W_fc, b_fc, W_dir, b_dir, W_dist, b_dist):
    bm = 512
    grid = (B // bm,)
    return pl.pallas_call(
        _head_body,
        grid=grid,
        in_specs=[
            pl.BlockSpec((bm, D), lambda i: (i, 0)),
            pl.BlockSpec((bm, D), lambda i: (i, 0)),
            pl.BlockSpec((2 * D, HIDDEN), lambda i: (0, 0)),
            pl.BlockSpec((1, HIDDEN), lambda i: (0, 0)),
            pl.BlockSpec((HIDDEN, VOCAB), lambda i: (0, 0)),
            pl.BlockSpec((1, VOCAB), lambda i: (0, 0)),
            pl.BlockSpec((HIDDEN, 1), lambda i: (0, 0)),
            pl.BlockSpec((1, 1), lambda i: (0, 0)),
        ],
        out_specs=[
            pl.BlockSpec((bm, VOCAB), lambda i: (i, 0)),
            pl.BlockSpec((bm, 1), lambda i: (i, 0)),
        ],
        out_shape=[
            jax.ShapeDtypeStruct((B, VOCAB), jnp.float32),
            jax.ShapeDtypeStruct((B, 1), jnp.float32),
        ],
    )(ne, fe, W_fc, b_fc, W_dir, b_dir, W_dist, b_dist)


def kernel(x, edge_index, edge_type, nest, food, W_rel, W_root, b_rgcn,
           W_fc, b_fc, W_dir, b_dir, W_dist, b_dist):
    src = edge_index[0].astype(jnp.int32)
    dst = edge_index[1].astype(jnp.int32)
    et = edge_type.astype(jnp.int32)
    nest32 = nest.astype(jnp.int32)
    food32 = food.astype(jnp.int32)

    epack = src + dst * 16384 + et * 268435456

    Y, root = _mm_call(x, W_rel, W_root)
    Y2 = Y.reshape(NREL * N, D)
    cnt2 = _cnt_call(epack)
    inv = _inv_call(cnt2.reshape(NC, NPAD // D, D)).reshape(NPAD)
    msgp = _main_call(Y2, epack, inv)
    ne, fe = _emb_call(root, msgp[0], msgp[1], b_rgcn, nest32, food32)
    la, tb = _head_call(ne, fe, W_fc, b_fc.reshape(1, -1),
                        W_dir, b_dir.reshape(1, -1),
                        W_dist, b_dist.reshape(1, -1))
    return (la, tb)


# speedup vs baseline: 19.0988x; 1.0302x over previous
import functools

import jax
import jax.numpy as jnp
from jax import lax
from jax.experimental import pallas as pl
from jax.experimental.pallas import tpu as pltpu
from jax.experimental.pallas import tpu_sc as plsc

N = 10000
E = 320000
D = 128
NREL = 4
B = 4096
HIDDEN = 256
VOCAB = 8

NC = 2
NS = 16
NW = NC * NS
EPW = E // NW
SUB = 80
GRP = 5
BLK = SUB * GRP
NBLK = EPW // BLK
NPAD = 40960
ZROWS = 624
CNT_SL = NPAD // NS

_mesh = plsc.VectorSubcoreMesh(core_axis_name="c", subcore_axis_name="s")


def _wid():
    return lax.axis_index("s") * NC + lax.axis_index("c")


def _mm_body(x_ref, wrel_ref, wroot_ref, y_ref, root_ref):
    xb = x_ref[...]
    for r in range(NREL):
        y_ref[r] = jnp.dot(xb, wrel_ref[r], preferred_element_type=jnp.float32)
    root_ref[...] = jnp.dot(xb, wroot_ref[...], preferred_element_type=jnp.float32)


def _mm_call(x, W_rel, W_root):
    bm = 400
    grid = (N // bm,)
    return pl.pallas_call(
        _mm_body,
        grid=grid,
        in_specs=[
            pl.BlockSpec((bm, D), lambda i: (i, 0)),
            pl.BlockSpec((NREL, D, D), lambda i: (0, 0, 0)),
            pl.BlockSpec((D, D), lambda i: (0, 0)),
        ],
        out_specs=[
            pl.BlockSpec((NREL, bm, D), lambda i: (0, i, 0)),
            pl.BlockSpec((bm, D), lambda i: (i, 0)),
        ],
        out_shape=[
            jax.ShapeDtypeStruct((NREL, N, D), jnp.float32),
            jax.ShapeDtypeStruct((N, D), jnp.float32),
        ],
    )(x, W_rel, W_root)


SUBC = 128
NFULL = EPW // SUBC
TAIL = EPW - NFULL * SUBC
MASK14 = 16383


def _unpack(p):
    sv = p & MASK14
    dv = (p >> 14) & MASK14
    tv = p >> 28
    return sv, dv, tv


def _cnt_body(ep_hbm, out_hbm, pbuf, sidx, sidx2, sidxt, ones, zbuf, semb0, semb1, acc):
    c = lax.axis_index("c")
    s = lax.axis_index("s")
    wid = _wid()

    pltpu.sync_copy(ep_hbm.at[pl.ds(wid * EPW, EPW)], pbuf)
    for k in range(SUBC // 16):
        ones[pl.ds(k * 16, 16)] = jnp.full((16,), 1.0, jnp.float32)

    def zb(i, _):
        zbuf[pl.ds(i * 16, 16)] = jnp.zeros((16,), jnp.float32)
        return 0
    lax.fori_loop(0, CNT_SL // 16, zb, 0)
    pltpu.sync_copy(zbuf, acc.at[pl.ds(s * CNT_SL, CNT_SL)])
    plsc.subcore_barrier()

    def sget(j, si):
        base = j * SUBC
        for m in range(SUBC // 16):
            p = pbuf[pl.ds(base + m * 16, 16)]
            _, dv, tv = _unpack(p)
            si[pl.ds(m * 16, 16)] = dv * NREL + tv

    sget(0, sidx)
    pltpu.async_copy(ones, acc.at[sidx], semb0, add=True)
    sget(1, sidx2)
    pltpu.async_copy(ones, acc.at[sidx2], semb1, add=True)

    def body(i, _):
        j0 = 2 * i
        pltpu.make_async_copy(ones, acc.at[sidx], semb0).wait()
        sget(j0 + 2, sidx)
        pltpu.async_copy(ones, acc.at[sidx], semb0, add=True)
        pltpu.make_async_copy(ones, acc.at[sidx2], semb1).wait()

        @pl.when(i < NFULL // 2 - 2)
        def _():
            sget(j0 + 3, sidx2)
            pltpu.async_copy(ones, acc.at[sidx2], semb1, add=True)
        return 0
    lax.fori_loop(0, NFULL // 2 - 1, body, 0)
    pltpu.make_async_copy(ones, acc.at[sidx], semb0).wait()
    sget(NFULL - 1, sidx2)
    pltpu.sync_copy(ones, acc.at[sidx2], add=True)
    p = pbuf[pl.ds(NFULL * SUBC, TAIL)]
    _, dv, tv = _unpack(p)
    sidxt[...] = dv * NREL + tv
    pltpu.sync_copy(ones.at[pl.ds(0, TAIL)], acc.at[sidxt], add=True)

    plsc.subcore_barrier()
    pltpu.sync_copy(acc.at[pl.ds(s * CNT_SL, CNT_SL)],
                    out_hbm.at[c, pl.ds(s * CNT_SL, CNT_SL)])


_cnt_call = functools.partial(
    pl.kernel,
    out_type=jax.ShapeDtypeStruct((NC, NPAD), jnp.float32),
    mesh=_mesh,
    compiler_params=pltpu.CompilerParams(needs_layout_passes=False),
    scratch_types=[
        pltpu.VMEM((EPW,), jnp.int32),
        pltpu.VMEM((SUBC,), jnp.int32),
        pltpu.VMEM((SUBC,), jnp.int32),
        pltpu.VMEM((TAIL,), jnp.int32),
        pltpu.VMEM((SUBC,), jnp.float32),
        pltpu.VMEM((CNT_SL,), jnp.float32),
        pltpu.SemaphoreType.DMA,
        pltpu.SemaphoreType.DMA,
        pltpu.VMEM_SHARED((NPAD,), jnp.float32),
    ],
)(_cnt_body)


def _inv_body(cnt_ref, inv_ref):
    cb = cnt_ref[...]
    inv_ref[...] = 1.0 / jnp.maximum(cb[0] + cb[1], 1.0)


def _inv_call(cnt3):
    nr = NPAD // D
    return pl.pallas_call(
        _inv_body,
        grid=(1,),
        in_specs=[pl.BlockSpec((NC, nr, D), lambda i: (0, 0, 0))],
        out_specs=pl.BlockSpec((nr, D), lambda i: (0, 0)),
        out_shape=jax.ShapeDtypeStruct((nr, D), jnp.float32),
    )(cnt3)


NITER = NFULL // 2


def _main_body(y_hbm, ep_hbm, inv_hbm, out_hbm,
               pbuf, rows0, rows1, scale0, scale1,
               gidx0, didx0, sidx0, gidx1, didx1, sidx1,
               gidxt, didxt, sidxt,
               semg0, semg1, sems0, sems1, acc):
    c = lax.axis_index("c")
    s = lax.axis_index("s")
    wid = _wid()

    pltpu.sync_copy(ep_hbm.at[pl.ds(wid * EPW, EPW)], pbuf)

    def zrow(r, _):
        for cc in range(D // 16):
            rows0[r, pl.ds(cc * 16, 16)] = jnp.zeros((16,), jnp.float32)
        return 0
    lax.fori_loop(0, SUBC, zrow, 0)
    z0 = s * ZROWS
    for q in range(ZROWS // SUBC):
        pltpu.sync_copy(rows0, acc.at[pl.ds(z0 + q * SUBC, SUBC)])
    rem = ZROWS - (ZROWS // SUBC) * SUBC
    pltpu.sync_copy(rows0.at[pl.ds(0, rem)],
                    acc.at[pl.ds(z0 + (ZROWS // SUBC) * SUBC, rem)])

    @pl.when(s == NS - 1)
    def _():
        pltpu.sync_copy(rows0.at[pl.ds(0, N - NS * ZROWS)],
                        acc.at[pl.ds(NS * ZROWS, N - NS * ZROWS)])
    plsc.subcore_barrier()

    def prep(j, gi, di, si, rw, sc, sg):
        base = j * SUBC
        for m in range(SUBC // 16):
            p = pbuf[pl.ds(base + m * 16, 16)]
            sv, dv, tv = _unpack(p)
            gi[pl.ds(m * 16, 16)] = tv * N + sv
            di[pl.ds(m * 16, 16)] = dv
            si[pl.ds(m * 16, 16)] = dv * NREL + tv
        pltpu.async_copy(y_hbm.at[gi], rw, sg)
        pltpu.async_copy(inv_hbm.at[si], sc, sg)

    def proc(gi, di, si, rw, sc, sg, ss):
        pltpu.make_async_copy(y_hbm.at[gi], rw, sg).wait()
        pltpu.make_async_copy(inv_hbm.at[si], sc, sg).wait()

        def mrow(r, _):
            s16 = plsc.load_gather(sc, [jnp.full((16,), r, jnp.int32)])
            for cc in range(D // 16):
                rw[r, pl.ds(cc * 16, 16)] = rw[r, pl.ds(cc * 16, 16)] * s16
            return 0
        lax.fori_loop(0, SUBC, mrow, 0, unroll=4)
        pltpu.async_copy(rw, acc.at[di], ss, add=True)

    def drain(rw, di, ss):
        pltpu.make_async_copy(rw, acc.at[di], ss).wait()

    set0 = (gidx0, didx0, sidx0, rows0, scale0, semg0)
    set1 = (gidx1, didx1, sidx1, rows1, scale1, semg1)

    prep(0, *set0)
    prep(1, *set1)

    def body(i, _):
        j0 = 2 * i
        proc(*set0[:5], semg0, sems0)
        proc(*set1[:5], semg1, sems1)

        @pl.when(i < NITER - 1)
        def _():
            drain(rows0, didx0, sems0)
            prep(j0 + 2, *set0)
            drain(rows1, didx1, sems1)
            prep(j0 + 3, *set1)
        return 0
    lax.fori_loop(0, NITER, body, 0)
    drain(rows0, didx0, sems0)
    drain(rows1, didx1, sems1)

    p = pbuf[pl.ds(NFULL * SUBC, TAIL)]
    sv, dv, tv = _unpack(p)
    gidxt[...] = tv * N + sv
    didxt[...] = dv
    sidxt[...] = dv * NREL + tv
    cp1 = pltpu.async_copy(y_hbm.at[gidxt], rows0.at[pl.ds(0, TAIL)], semg0)
    cp2 = pltpu.async_copy(inv_hbm.at[sidxt], scale0.at[pl.ds(0, TAIL)], semg0)
    cp1.wait()
    cp2.wait()

    def mrowt(r, _):
        s16 = plsc.load_gather(scale0, [jnp.full((16,), r, jnp.int32)])
        for cc in range(D // 16):
            rows0[r, pl.ds(cc * 16, 16)] = rows0[r, pl.ds(cc * 16, 16)] * s16
        return 0
    lax.fori_loop(0, TAIL, mrowt, 0)
    pltpu.sync_copy(rows0.at[pl.ds(0, TAIL)], acc.at[didxt], add=True)

    plsc.subcore_barrier()
    r0 = s * ZROWS
    for q in range(ZROWS // SUBC):
        pltpu.sync_copy(acc.at[pl.ds(r0 + q * SUBC, SUBC)],
                        out_hbm.at[c, pl.ds(r0 + q * SUBC, SUBC)])
    pltpu.sync_copy(acc.at[pl.ds(r0 + (ZROWS // SUBC) * SUBC, rem)],
                    out_hbm.at[c, pl.ds(r0 + (ZROWS // SUBC) * SUBC, rem)])

    @pl.when(s == NS - 1)
    def _():
        pltpu.sync_copy(acc.at[pl.ds(NS * ZROWS, N - NS * ZROWS)],
                        out_hbm.at[c, pl.ds(NS * ZROWS, N - NS * ZROWS)])


_main_call = functools.partial(
    pl.kernel,
    out_type=jax.ShapeDtypeStruct((NC, N, D), jnp.float32),
    mesh=_mesh,
    compiler_params=pltpu.CompilerParams(needs_layout_passes=False),
    scratch_types=[
        pltpu.VMEM((EPW,), jnp.int32),
        pltpu.VMEM((SUBC, D), jnp.float32),
        pltpu.VMEM((SUBC, D), jnp.float32),
        pltpu.VMEM((SUBC,), jnp.float32),
        pltpu.VMEM((SUBC,), jnp.float32),
        pltpu.VMEM((SUBC,), jnp.int32),
        pltpu.VMEM((SUBC,), jnp.int32),
        pltpu.VMEM((SUBC,), jnp.int32),
        pltpu.VMEM((SUBC,), jnp.int32),
        pltpu.VMEM((SUBC,), jnp.int32),
        pltpu.VMEM((SUBC,), jnp.int32),
        pltpu.VMEM((TAIL,), jnp.int32),
        pltpu.VMEM((TAIL,), jnp.int32),
        pltpu.VMEM((TAIL,), jnp.int32),
        pltpu.SemaphoreType.DMA,
        pltpu.SemaphoreType.DMA,
        pltpu.SemaphoreType.DMA,
        pltpu.SemaphoreType.DMA,
        pltpu.VMEM_SHARED((N, D), jnp.float32),
    ],
)(_main_body)


_EPT = B // NW


def _emb_body(root_hbm, m0_hbm, m1_hbm, b_hbm, nest_hbm, food_hbm,
              ne_hbm, fe_hbm, idxn, idxf, rn, m0n, m1n, rf, m0f, m1f,
              bbuf, semn, semf):
    wid = _wid()
    base = wid * _EPT
    pltpu.sync_copy(nest_hbm.at[pl.ds(base, _EPT)], idxn)
    pltpu.sync_copy(food_hbm.at[pl.ds(base, _EPT)], idxf)
    pltpu.sync_copy(b_hbm, bbuf)
    pltpu.async_copy(root_hbm.at[idxn], rn, semn)
    pltpu.async_copy(m0_hbm.at[idxn], m0n, semn)
    pltpu.async_copy(m1_hbm.at[idxn], m1n, semn)
    pltpu.async_copy(root_hbm.at[idxf], rf, semf)
    pltpu.async_copy(m0_hbm.at[idxf], m0f, semf)
    pltpu.async_copy(m1_hbm.at[idxf], m1f, semf)

    def combine(rb, m0b, m1b):
        def row(r, _):
            for cc in range(D // 16):
                ds = pl.ds(cc * 16, 16)
                v = rb[r, ds] + m0b[r, ds] + m1b[r, ds] + bbuf[ds]
                rb[r, ds] = jnp.maximum(v, 0.0)
            return 0
        lax.fori_loop(0, _EPT, row, 0, unroll=4)

    pltpu.make_async_copy(root_hbm.at[idxn], rn, semn).wait()
    pltpu.make_async_copy(m0_hbm.at[idxn], m0n, semn).wait()
    pltpu.make_async_copy(m1_hbm.at[idxn], m1n, semn).wait()
    combine(rn, m0n, m1n)
    pltpu.async_copy(rn, ne_hbm.at[pl.ds(base, _EPT)], semn)

    pltpu.make_async_copy(root_hbm.at[idxf], rf, semf).wait()
    pltpu.make_async_copy(m0_hbm.at[idxf], m0f, semf).wait()
    pltpu.make_async_copy(m1_hbm.at[idxf], m1f, semf).wait()
    combine(rf, m0f, m1f)
    pltpu.async_copy(rf, fe_hbm.at[pl.ds(base, _EPT)], semf)

    pltpu.make_async_copy(rn, ne_hbm.at[pl.ds(base, _EPT)], semn).wait()
    pltpu.make_async_copy(rf, fe_hbm.at[pl.ds(base, _EPT)], semf).wait()


_emb_call = functools.partial(
    pl.kernel,
    out_type=[jax.ShapeDtypeStruct((B, D), jnp.float32),
              jax.ShapeDtypeStruct((B, D), jnp.float32)],
    mesh=_mesh,
    compiler_params=pltpu.CompilerParams(needs_layout_passes=False),
    scratch_types=[
        pltpu.VMEM((_EPT,), jnp.int32),
        pltpu.VMEM((_EPT,), jnp.int32),
        pltpu.VMEM((_EPT, D), jnp.float32),
        pltpu.VMEM((_EPT, D), jnp.float32),
        pltpu.VMEM((_EPT, D), jnp.float32),
        pltpu.VMEM((_EPT, D), jnp.float32),
        pltpu.VMEM((_EPT, D), jnp.float32),
        pltpu.VMEM((_EPT, D), jnp.float32),
        pltpu.VMEM((D,), jnp.float32),
        pltpu.SemaphoreType.DMA,
        pltpu.SemaphoreType.DMA,
    ],
)(_emb_body)


def _head_body(ne_ref, fe_ref, wfc_ref, bfc_ref, wdir_ref, bdir_ref,
               wdist_ref, bdist_ref, la_ref, tb_ref):
    hid = jnp.dot(ne_ref[...], wfc_ref[:D], preferred_element_type=jnp.float32)
    hid = hid + jnp.dot(fe_ref[...], wfc_ref[D:],
                        preferred_element_type=jnp.float32)
    hid = jnp.maximum(hid + bfc_ref[...], 0.0)
    logit = jnp.dot(hid, wdir_ref[...], preferred_element_type=jnp.float32)
    logit = logit + bdir_ref[...]
    m = jnp.max(logit, axis=-1, keepdims=True)
    lse = jnp.log(jnp.sum(jnp.exp(logit - m), axis=-1, keepdims=True)) + m
    la_ref[...] = logit - lse
    tb_ref[...] = (jnp.dot(hid, wdist_ref[...],
                           preferred_element_type=jnp.float32)
                   + bdist_ref[...])


def _head_call(ne, fe, W_fc, b_fc, W_dir, b_dir, W_dist, b_dist):
    bm = 512
    grid = (B // bm,)
    return pl.pallas_call(
        _head_body,
        grid=grid,
        in_specs=[
            pl.BlockSpec((bm, D), lambda i: (i, 0)),
            pl.BlockSpec((bm, D), lambda i: (i, 0)),
            pl.BlockSpec((2 * D, HIDDEN), lambda i: (0, 0)),
            pl.BlockSpec((1, HIDDEN), lambda i: (0, 0)),
            pl.BlockSpec((HIDDEN, VOCAB), lambda i: (0, 0)),
            pl.BlockSpec((1, VOCAB), lambda i: (0, 0)),
            pl.BlockSpec((HIDDEN, 1), lambda i: (0, 0)),
            pl.BlockSpec((1, 1), lambda i: (0, 0)),
        ],
        out_specs=[
            pl.BlockSpec((bm, VOCAB), lambda i: (i, 0)),
            pl.BlockSpec((bm, 1), lambda i: (i, 0)),
        ],
        out_shape=[
            jax.ShapeDtypeStruct((B, VOCAB), jnp.float32),
            jax.ShapeDtypeStruct((B, 1), jnp.float32),
        ],
    )(ne, fe, W_fc, b_fc, W_dir, b_dir, W_dist, b_dist)


def kernel(x, edge_index, edge_type, nest, food, W_rel, W_root, b_rgcn,
           W_fc, b_fc, W_dir, b_dir, W_dist, b_dist):
    src = edge_index[0].astype(jnp.int32)
    dst = edge_index[1].astype(jnp.int32)
    et = edge_type.astype(jnp.int32)
    nest32 = nest.astype(jnp.int32)
    food32 = food.astype(jnp.int32)

    epack = src + dst * 16384 + et * 268435456

    Y, root = _mm_call(x, W_rel, W_root)
    Y2 = Y.reshape(NREL * N, D)
    cnt2 = _cnt_call(epack)
    inv = _inv_call(cnt2.reshape(NC, NPAD // D, D)).reshape(NPAD)
    msgp = _main_call(Y2, epack, inv)
    ne, fe = _emb_call(root, msgp[0], msgp[1], b_rgcn, nest32, food32)
    la, tb = _head_call(ne, fe, W_fc, b_fc.reshape(1, -1),
                        W_dir, b_dir.reshape(1, -1),
                        W_dist, b_dist.reshape(1, -1))
    return (la, tb)


# speedup vs baseline: 19.1973x; 1.0052x over previous
import functools

import jax
import jax.numpy as jnp
from jax import lax
from jax.experimental import pallas as pl
from jax.experimental.pallas import tpu as pltpu
from jax.experimental.pallas import tpu_sc as plsc

N = 10000
E = 320000
D = 128
NREL = 4
B = 4096
HIDDEN = 256
VOCAB = 8

NC = 2
NS = 16
NW = NC * NS
EPW = E // NW
SUB = 80
GRP = 5
BLK = SUB * GRP
NBLK = EPW // BLK
NPAD = 40960
ZROWS = 624
CNT_SL = NPAD // NS

_mesh = plsc.VectorSubcoreMesh(core_axis_name="c", subcore_axis_name="s")


def _wid():
    return lax.axis_index("s") * NC + lax.axis_index("c")


def _mm_body(x_ref, wrel_ref, wroot_ref, y_ref, root_ref):
    xb = x_ref[...]
    for r in range(NREL):
        y_ref[r] = jnp.dot(xb, wrel_ref[r], preferred_element_type=jnp.float32)
    root_ref[...] = jnp.dot(xb, wroot_ref[...], preferred_element_type=jnp.float32)


def _mm_call(x, W_rel, W_root):
    bm = 400
    grid = (N // bm,)
    return pl.pallas_call(
        _mm_body,
        grid=grid,
        in_specs=[
            pl.BlockSpec((bm, D), lambda i: (i, 0)),
            pl.BlockSpec((NREL, D, D), lambda i: (0, 0, 0)),
            pl.BlockSpec((D, D), lambda i: (0, 0)),
        ],
        out_specs=[
            pl.BlockSpec((NREL, bm, D), lambda i: (0, i, 0)),
            pl.BlockSpec((bm, D), lambda i: (i, 0)),
        ],
        out_shape=[
            jax.ShapeDtypeStruct((NREL, N, D), jnp.float32),
            jax.ShapeDtypeStruct((N, D), jnp.float32),
        ],
    )(x, W_rel, W_root)


SUBC = 128
NFULL = EPW // SUBC
TAIL = EPW - NFULL * SUBC
MASK14 = 16383


def _unpack(p):
    sv = p & MASK14
    dv = (p >> 14) & MASK14
    tv = p >> 28
    return sv, dv, tv


def _cnt_body(sb_hbm, db_hbm, tb_hbm, cnt_hbm, ep_hbm,
              sbuf, dbuf, tbuf, ebuf, sidx, sidx2, sidxt, ones, zbuf,
              semb0, semb1, acc):
    c = lax.axis_index("c")
    s = lax.axis_index("s")
    wid = _wid()

    pltpu.sync_copy(sb_hbm.at[pl.ds(wid * EPW, EPW)], sbuf)
    pltpu.sync_copy(db_hbm.at[pl.ds(wid * EPW, EPW)], dbuf)
    pltpu.sync_copy(tb_hbm.at[pl.ds(wid * EPW, EPW)], tbuf)

    def pk(i, _):
        sv = sbuf[pl.ds(i * 16, 16)]
        dv = dbuf[pl.ds(i * 16, 16)]
        tv = tbuf[pl.ds(i * 16, 16)]
        ebuf[pl.ds(i * 16, 16)] = sv + dv * 16384 + tv * 268435456
        return 0
    lax.fori_loop(0, EPW // 16, pk, 0, unroll=4)
    pltpu.sync_copy(ebuf, ep_hbm.at[pl.ds(wid * EPW, EPW)])

    for k in range(SUBC // 16):
        ones[pl.ds(k * 16, 16)] = jnp.full((16,), 1.0, jnp.float32)

    def zb(i, _):
        zbuf[pl.ds(i * 16, 16)] = jnp.zeros((16,), jnp.float32)
        return 0
    lax.fori_loop(0, CNT_SL // 16, zb, 0)
    pltpu.sync_copy(zbuf, acc.at[pl.ds(s * CNT_SL, CNT_SL)])
    plsc.subcore_barrier()

    def sget(j, si):
        base = j * SUBC
        for m in range(SUBC // 16):
            dv = dbuf[pl.ds(base + m * 16, 16)]
            tv = tbuf[pl.ds(base + m * 16, 16)]
            si[pl.ds(m * 16, 16)] = dv * NREL + tv

    sget(0, sidx)
    pltpu.async_copy(ones, acc.at[sidx], semb0, add=True)
    sget(1, sidx2)
    pltpu.async_copy(ones, acc.at[sidx2], semb1, add=True)

    def body(i, _):
        j0 = 2 * i
        pltpu.make_async_copy(ones, acc.at[sidx], semb0).wait()
        sget(j0 + 2, sidx)
        pltpu.async_copy(ones, acc.at[sidx], semb0, add=True)
        pltpu.make_async_copy(ones, acc.at[sidx2], semb1).wait()

        @pl.when(i < NFULL // 2 - 2)
        def _():
            sget(j0 + 3, sidx2)
            pltpu.async_copy(ones, acc.at[sidx2], semb1, add=True)
        return 0
    lax.fori_loop(0, NFULL // 2 - 1, body, 0)
    pltpu.make_async_copy(ones, acc.at[sidx], semb0).wait()
    sget(NFULL - 1, sidx2)
    pltpu.sync_copy(ones, acc.at[sidx2], add=True)
    dv = dbuf[pl.ds(NFULL * SUBC, TAIL)]
    tv = tbuf[pl.ds(NFULL * SUBC, TAIL)]
    sidxt[...] = dv * NREL + tv
    pltpu.sync_copy(ones.at[pl.ds(0, TAIL)], acc.at[sidxt], add=True)

    plsc.subcore_barrier()
    pltpu.sync_copy(acc.at[pl.ds(s * CNT_SL, CNT_SL)],
                    cnt_hbm.at[c, pl.ds(s * CNT_SL, CNT_SL)])


_cnt_call = functools.partial(
    pl.kernel,
    out_type=[jax.ShapeDtypeStruct((NC, NPAD), jnp.float32),
              jax.ShapeDtypeStruct((E,), jnp.int32)],
    mesh=_mesh,
    compiler_params=pltpu.CompilerParams(needs_layout_passes=False),
    scratch_types=[
        pltpu.VMEM((EPW,), jnp.int32),
        pltpu.VMEM((EPW,), jnp.int32),
        pltpu.VMEM((EPW,), jnp.int32),
        pltpu.VMEM((EPW,), jnp.int32),
        pltpu.VMEM((SUBC,), jnp.int32),
        pltpu.VMEM((SUBC,), jnp.int32),
        pltpu.VMEM((TAIL,), jnp.int32),
        pltpu.VMEM((SUBC,), jnp.float32),
        pltpu.VMEM((CNT_SL,), jnp.float32),
        pltpu.SemaphoreType.DMA,
        pltpu.SemaphoreType.DMA,
        pltpu.VMEM_SHARED((NPAD,), jnp.float32),
    ],
)(_cnt_body)


def _inv_body(cnt_ref, inv_ref):
    cb = cnt_ref[...]
    inv_ref[...] = 1.0 / jnp.maximum(cb[0] + cb[1], 1.0)


def _inv_call(cnt3):
    nr = NPAD // D
    return pl.pallas_call(
        _inv_body,
        grid=(1,),
        in_specs=[pl.BlockSpec((NC, nr, D), lambda i: (0, 0, 0))],
        out_specs=pl.BlockSpec((nr, D), lambda i: (0, 0)),
        out_shape=jax.ShapeDtypeStruct((nr, D), jnp.float32),
    )(cnt3)


NITER = NFULL // 2


def _main_body(y_hbm, ep_hbm, inv_hbm, out0_hbm, out1_hbm,
               pbuf, rows0, rows1, scale0, scale1,
               gidx0, didx0, sidx0, gidx1, didx1, sidx1,
               gidxt, didxt, sidxt,
               semg0, semg1, sems0, sems1, acc):
    c = lax.axis_index("c")
    s = lax.axis_index("s")
    wid = _wid()

    pltpu.sync_copy(ep_hbm.at[pl.ds(wid * EPW, EPW)], pbuf)

    def zrow(r, _):
        for cc in range(D // 16):
            rows0[r, pl.ds(cc * 16, 16)] = jnp.zeros((16,), jnp.float32)
        return 0
    lax.fori_loop(0, SUBC, zrow, 0)
    z0 = s * ZROWS
    for q in range(ZROWS // SUBC):
        pltpu.sync_copy(rows0, acc.at[pl.ds(z0 + q * SUBC, SUBC)])
    rem = ZROWS - (ZROWS // SUBC) * SUBC
    pltpu.sync_copy(rows0.at[pl.ds(0, rem)],
                    acc.at[pl.ds(z0 + (ZROWS // SUBC) * SUBC, rem)])

    @pl.when(s == NS - 1)
    def _():
        pltpu.sync_copy(rows0.at[pl.ds(0, N - NS * ZROWS)],
                        acc.at[pl.ds(NS * ZROWS, N - NS * ZROWS)])
    plsc.subcore_barrier()

    def prep(j, gi, di, si, rw, sc, sg):
        base = j * SUBC
        for m in range(SUBC // 16):
            p = pbuf[pl.ds(base + m * 16, 16)]
            sv, dv, tv = _unpack(p)
            gi[pl.ds(m * 16, 16)] = tv * N + sv
            di[pl.ds(m * 16, 16)] = dv
            si[pl.ds(m * 16, 16)] = dv * NREL + tv
        pltpu.async_copy(y_hbm.at[gi], rw, sg)
        pltpu.async_copy(inv_hbm.at[si], sc, sg)

    def proc(gi, di, si, rw, sc, sg, ss):
        pltpu.make_async_copy(y_hbm.at[gi], rw, sg).wait()
        pltpu.make_async_copy(inv_hbm.at[si], sc, sg).wait()

        def mrow(r, _):
            s16 = plsc.load_gather(sc, [jnp.full((16,), r, jnp.int32)])
            for cc in range(D // 16):
                rw[r, pl.ds(cc * 16, 16)] = rw[r, pl.ds(cc * 16, 16)] * s16
            return 0
        lax.fori_loop(0, SUBC, mrow, 0, unroll=4)
        pltpu.async_copy(rw, acc.at[di], ss, add=True)

    def drain(rw, di, ss):
        pltpu.make_async_copy(rw, acc.at[di], ss).wait()

    set0 = (gidx0, didx0, sidx0, rows0, scale0, semg0)
    set1 = (gidx1, didx1, sidx1, rows1, scale1, semg1)

    prep(0, *set0)
    prep(1, *set1)

    def body(i, _):
        j0 = 2 * i
        proc(*set0[:5], semg0, sems0)
        proc(*set1[:5], semg1, sems1)

        @pl.when(i < NITER - 1)
        def _():
            drain(rows0, didx0, sems0)
            prep(j0 + 2, *set0)
            drain(rows1, didx1, sems1)
            prep(j0 + 3, *set1)
        return 0
    lax.fori_loop(0, NITER, body, 0)
    drain(rows0, didx0, sems0)
    drain(rows1, didx1, sems1)

    p = pbuf[pl.ds(NFULL * SUBC, TAIL)]
    sv, dv, tv = _unpack(p)
    gidxt[...] = tv * N + sv
    didxt[...] = dv
    sidxt[...] = dv * NREL + tv
    cp1 = pltpu.async_copy(y_hbm.at[gidxt], rows0.at[pl.ds(0, TAIL)], semg0)
    cp2 = pltpu.async_copy(inv_hbm.at[sidxt], scale0.at[pl.ds(0, TAIL)], semg0)
    cp1.wait()
    cp2.wait()

    def mrowt(r, _):
        s16 = plsc.load_gather(scale0, [jnp.full((16,), r, jnp.int32)])
        for cc in range(D // 16):
            rows0[r, pl.ds(cc * 16, 16)] = rows0[r, pl.ds(cc * 16, 16)] * s16
        return 0
    lax.fori_loop(0, TAIL, mrowt, 0)
    pltpu.sync_copy(rows0.at[pl.ds(0, TAIL)], acc.at[didxt], add=True)

    plsc.subcore_barrier()
    r0 = s * ZROWS
    for cc_, o_hbm in ((0, out0_hbm), (1, out1_hbm)):
        @pl.when(c == cc_)
        def _(o_hbm=o_hbm):
            for q in range(ZROWS // SUBC):
                pltpu.sync_copy(acc.at[pl.ds(r0 + q * SUBC, SUBC)],
                                o_hbm.at[pl.ds(r0 + q * SUBC, SUBC)])
            pltpu.sync_copy(acc.at[pl.ds(r0 + (ZROWS // SUBC) * SUBC, rem)],
                            o_hbm.at[pl.ds(r0 + (ZROWS // SUBC) * SUBC, rem)])

            @pl.when(s == NS - 1)
            def _():
                pltpu.sync_copy(acc.at[pl.ds(NS * ZROWS, N - NS * ZROWS)],
                                o_hbm.at[pl.ds(NS * ZROWS, N - NS * ZROWS)])


_main_call = functools.partial(
    pl.kernel,
    out_type=[jax.ShapeDtypeStruct((N, D), jnp.float32),
              jax.ShapeDtypeStruct((N, D), jnp.float32)],
    mesh=_mesh,
    compiler_params=pltpu.CompilerParams(needs_layout_passes=False),
    scratch_types=[
        pltpu.VMEM((EPW,), jnp.int32),
        pltpu.VMEM((SUBC, D), jnp.float32),
        pltpu.VMEM((SUBC, D), jnp.float32),
        pltpu.VMEM((SUBC,), jnp.float32),
        pltpu.VMEM((SUBC,), jnp.float32),
        pltpu.VMEM((SUBC,), jnp.int32),
        pltpu.VMEM((SUBC,), jnp.int32),
        pltpu.VMEM((SUBC,), jnp.int32),
        pltpu.VMEM((SUBC,), jnp.int32),
        pltpu.VMEM((SUBC,), jnp.int32),
        pltpu.VMEM((SUBC,), jnp.int32),
        pltpu.VMEM((TAIL,), jnp.int32),
        pltpu.VMEM((TAIL,), jnp.int32),
        pltpu.VMEM((TAIL,), jnp.int32),
        pltpu.SemaphoreType.DMA,
        pltpu.SemaphoreType.DMA,
        pltpu.SemaphoreType.DMA,
        pltpu.SemaphoreType.DMA,
        pltpu.VMEM_SHARED((N, D), jnp.float32),
    ],
)(_main_body)


_EPT = B // NW


def _emb_body(root_hbm, m0_hbm, m1_hbm, b_hbm, nest_hbm, food_hbm,
              ne_hbm, fe_hbm, idxn, idxf, rn, m0n, m1n, rf, m0f, m1f,
              bbuf, semn, semf):
    wid = _wid()
    base = wid * _EPT
    pltpu.sync_copy(nest_hbm.at[pl.ds(base, _EPT)], idxn)
    pltpu.sync_copy(food_hbm.at[pl.ds(base, _EPT)], idxf)
    pltpu.sync_copy(b_hbm, bbuf)
    pltpu.async_copy(root_hbm.at[idxn], rn, semn)
    pltpu.async_copy(m0_hbm.at[idxn], m0n, semn)
    pltpu.async_copy(m1_hbm.at[idxn], m1n, semn)
    pltpu.async_copy(root_hbm.at[idxf], rf, semf)
    pltpu.async_copy(m0_hbm.at[idxf], m0f, semf)
    pltpu.async_copy(m1_hbm.at[idxf], m1f, semf)

    def combine(rb, m0b, m1b):
        def row(r, _):
            for cc in range(D // 16):
                ds = pl.ds(cc * 16, 16)
                v = rb[r, ds] + m0b[r, ds] + m1b[r, ds] + bbuf[ds]
                rb[r, ds] = jnp.maximum(v, 0.0)
            return 0
        lax.fori_loop(0, _EPT, row, 0, unroll=4)

    pltpu.make_async_copy(root_hbm.at[idxn], rn, semn).wait()
    pltpu.make_async_copy(m0_hbm.at[idxn], m0n, semn).wait()
    pltpu.make_async_copy(m1_hbm.at[idxn], m1n, semn).wait()
    combine(rn, m0n, m1n)
    pltpu.async_copy(rn, ne_hbm.at[pl.ds(base, _EPT)], semn)

    pltpu.make_async_copy(root_hbm.at[idxf], rf, semf).wait()
    pltpu.make_async_copy(m0_hbm.at[idxf], m0f, semf).wait()
    pltpu.make_async_copy(m1_hbm.at[idxf], m1f, semf).wait()
    combine(rf, m0f, m1f)
    pltpu.async_copy(rf, fe_hbm.at[pl.ds(base, _EPT)], semf)

    pltpu.make_async_copy(rn, ne_hbm.at[pl.ds(base, _EPT)], semn).wait()
    pltpu.make_async_copy(rf, fe_hbm.at[pl.ds(base, _EPT)], semf).wait()


_emb_call = functools.partial(
    pl.kernel,
    out_type=[jax.ShapeDtypeStruct((B, D), jnp.float32),
              jax.ShapeDtypeStruct((B, D), jnp.float32)],
    mesh=_mesh,
    compiler_params=pltpu.CompilerParams(needs_layout_passes=False),
    scratch_types=[
        pltpu.VMEM((_EPT,), jnp.int32),
        pltpu.VMEM((_EPT,), jnp.int32),
        pltpu.VMEM((_EPT, D), jnp.float32),
        pltpu.VMEM((_EPT, D), jnp.float32),
        pltpu.VMEM((_EPT, D), jnp.float32),
        pltpu.VMEM((_EPT, D), jnp.float32),
        pltpu.VMEM((_EPT, D), jnp.float32),
        pltpu.VMEM((_EPT, D), jnp.float32),
        pltpu.VMEM((D,), jnp.float32),
        pltpu.SemaphoreType.DMA,
        pltpu.SemaphoreType.DMA,
    ],
)(_emb_body)


def _head_body(ne_ref, fe_ref, wfc_ref, bfc_ref, whd_ref, bhd_ref,
               la_ref, tb_ref):
    hid = jnp.dot(ne_ref[...], wfc_ref[:D], preferred_element_type=jnp.float32)
    hid = hid + jnp.dot(fe_ref[...], wfc_ref[D:],
                        preferred_element_type=jnp.float32)
    hid = jnp.maximum(hid + bfc_ref[...], 0.0)
    z = jnp.dot(hid, whd_ref[...], preferred_element_type=jnp.float32)
    z = z + bhd_ref[...]
    logit = z[:, :VOCAB]
    m = jnp.max(logit, axis=-1, keepdims=True)
    lse = jnp.log(jnp.sum(jnp.exp(logit - m), axis=-1, keepdims=True)) + m
    la_ref[...] = logit - lse
    tb_ref[...] = z[:, VOCAB:]


def _head_call(ne, fe, W_fc, b_fc, W_hd, b_hd):
    bm = 512
    grid = (B // bm,)
    return pl.pallas_call(
        _head_body,
        grid=grid,
        in_specs=[
            pl.BlockSpec((bm, D), lambda i: (i, 0)),
            pl.BlockSpec((bm, D), lambda i: (i, 0)),
            pl.BlockSpec((2 * D, HIDDEN), lambda i: (0, 0)),
            pl.BlockSpec((1, HIDDEN), lambda i: (0, 0)),
            pl.BlockSpec((HIDDEN, VOCAB + 1), lambda i: (0, 0)),
            pl.BlockSpec((1, VOCAB + 1), lambda i: (0, 0)),
        ],
        out_specs=[
            pl.BlockSpec((bm, VOCAB), lambda i: (i, 0)),
            pl.BlockSpec((bm, 1), lambda i: (i, 0)),
        ],
        out_shape=[
            jax.ShapeDtypeStruct((B, VOCAB), jnp.float32),
            jax.ShapeDtypeStruct((B, 1), jnp.float32),
        ],
    )(ne, fe, W_fc, b_fc, W_hd, b_hd)


def kernel(x, edge_index, edge_type, nest, food, W_rel, W_root, b_rgcn,
           W_fc, b_fc, W_dir, b_dir, W_dist, b_dist):
    src = edge_index[0].astype(jnp.int32)
    dst = edge_index[1].astype(jnp.int32)
    et = edge_type.astype(jnp.int32)
    nest32 = nest.astype(jnp.int32)
    food32 = food.astype(jnp.int32)

    Y, root = _mm_call(x, W_rel, W_root)
    Y2 = Y.reshape(NREL * N, D)
    cnt2, epack = _cnt_call(src, dst, et)
    inv = _inv_call(cnt2.reshape(NC, NPAD // D, D)).reshape(NPAD)
    msg0, msg1 = _main_call(Y2, epack, inv)
    ne, fe = _emb_call(root, msg0, msg1, b_rgcn, nest32, food32)
    W_hd = jnp.concatenate([W_dir, W_dist], axis=1)
    b_hd = jnp.concatenate([b_dir, b_dist]).reshape(1, -1)
    la, tb = _head_call(ne, fe, W_fc, b_fc.reshape(1, -1), W_hd, b_hd)
    return (la, tb)


# speedup vs baseline: 19.9042x; 1.0368x over previous
import functools

import jax
import jax.numpy as jnp
from jax import lax
from jax.experimental import pallas as pl
from jax.experimental.pallas import tpu as pltpu
from jax.experimental.pallas import tpu_sc as plsc

N = 10000
E = 320000
D = 128
NREL = 4
B = 4096
HIDDEN = 256
VOCAB = 8

NC = 2
NS = 16
NW = NC * NS
EPW = E // NW
SUB = 80
GRP = 5
BLK = SUB * GRP
NBLK = EPW // BLK
NPAD = 40960
ZROWS = 624
CNT_SL = NPAD // NS

_mesh = plsc.VectorSubcoreMesh(core_axis_name="c", subcore_axis_name="s")


def _wid():
    return lax.axis_index("s") * NC + lax.axis_index("c")


def _mm_body(x_ref, wrel_ref, wroot_ref, b_ref, y_ref, root_ref):
    xb = x_ref[...]
    for r in range(NREL):
        y_ref[r] = jnp.dot(xb, wrel_ref[r], preferred_element_type=jnp.float32)
    root_ref[...] = (jnp.dot(xb, wroot_ref[...],
                             preferred_element_type=jnp.float32)
                     + b_ref[...])


def _mm_call(x, W_rel, W_root, b):
    bm = 400
    grid = (N // bm,)
    return pl.pallas_call(
        _mm_body,
        grid=grid,
        in_specs=[
            pl.BlockSpec((bm, D), lambda i: (i, 0)),
            pl.BlockSpec((NREL, D, D), lambda i: (0, 0, 0)),
            pl.BlockSpec((D, D), lambda i: (0, 0)),
            pl.BlockSpec((1, D), lambda i: (0, 0)),
        ],
        out_specs=[
            pl.BlockSpec((NREL, bm, D), lambda i: (0, i, 0)),
            pl.BlockSpec((bm, D), lambda i: (i, 0)),
        ],
        out_shape=[
            jax.ShapeDtypeStruct((NREL, N, D), jnp.float32),
            jax.ShapeDtypeStruct((N, D), jnp.float32),
        ],
    )(x, W_rel, W_root, b)


SUBC = 128
NFULL = EPW // SUBC
TAIL = EPW - NFULL * SUBC
MASK14 = 16383


def _unpack(p):
    sv = p & MASK14
    dv = (p >> 14) & MASK14
    tv = p >> 28
    return sv, dv, tv


def _cnt_body(sb_hbm, db_hbm, tb_hbm, cnt_hbm, ep_hbm,
              sbuf, dbuf, tbuf, ebuf, sidx, sidx2, sidxt, ones, zbuf,
              semb0, semb1, acc):
    c = lax.axis_index("c")
    s = lax.axis_index("s")
    wid = _wid()

    pltpu.sync_copy(sb_hbm.at[pl.ds(wid * EPW, EPW)], sbuf)
    pltpu.sync_copy(db_hbm.at[pl.ds(wid * EPW, EPW)], dbuf)
    pltpu.sync_copy(tb_hbm.at[pl.ds(wid * EPW, EPW)], tbuf)

    def pk(i, _):
        sv = sbuf[pl.ds(i * 16, 16)]
        dv = dbuf[pl.ds(i * 16, 16)]
        tv = tbuf[pl.ds(i * 16, 16)]
        ebuf[pl.ds(i * 16, 16)] = sv + dv * 16384 + tv * 268435456
        return 0
    lax.fori_loop(0, EPW // 16, pk, 0, unroll=4)
    pltpu.sync_copy(ebuf, ep_hbm.at[pl.ds(wid * EPW, EPW)])

    for k in range(SUBC // 16):
        ones[pl.ds(k * 16, 16)] = jnp.full((16,), 1.0, jnp.float32)

    def zb(i, _):
        zbuf[pl.ds(i * 16, 16)] = jnp.zeros((16,), jnp.float32)
        return 0
    lax.fori_loop(0, CNT_SL // 16, zb, 0)
    pltpu.sync_copy(zbuf, acc.at[pl.ds(s * CNT_SL, CNT_SL)])
    plsc.subcore_barrier()

    def sget(j, si):
        base = j * SUBC
        for m in range(SUBC // 16):
            dv = dbuf[pl.ds(base + m * 16, 16)]
            tv = tbuf[pl.ds(base + m * 16, 16)]
            si[pl.ds(m * 16, 16)] = dv * NREL + tv

    sget(0, sidx)
    pltpu.async_copy(ones, acc.at[sidx], semb0, add=True)
    sget(1, sidx2)
    pltpu.async_copy(ones, acc.at[sidx2], semb1, add=True)

    def body(i, _):
        j0 = 2 * i
        pltpu.make_async_copy(ones, acc.at[sidx], semb0).wait()
        sget(j0 + 2, sidx)
        pltpu.async_copy(ones, acc.at[sidx], semb0, add=True)
        pltpu.make_async_copy(ones, acc.at[sidx2], semb1).wait()

        @pl.when(i < NFULL // 2 - 2)
        def _():
            sget(j0 + 3, sidx2)
            pltpu.async_copy(ones, acc.at[sidx2], semb1, add=True)
        return 0
    lax.fori_loop(0, NFULL // 2 - 1, body, 0)
    pltpu.make_async_copy(ones, acc.at[sidx], semb0).wait()
    sget(NFULL - 1, sidx2)
    pltpu.sync_copy(ones, acc.at[sidx2], add=True)
    dv = dbuf[pl.ds(NFULL * SUBC, TAIL)]
    tv = tbuf[pl.ds(NFULL * SUBC, TAIL)]
    sidxt[...] = dv * NREL + tv
    pltpu.sync_copy(ones.at[pl.ds(0, TAIL)], acc.at[sidxt], add=True)

    plsc.subcore_barrier()
    pltpu.sync_copy(acc.at[pl.ds(s * CNT_SL, CNT_SL)],
                    cnt_hbm.at[c, pl.ds(s * CNT_SL, CNT_SL)])


_cnt_call = functools.partial(
    pl.kernel,
    out_type=[jax.ShapeDtypeStruct((NC, NPAD), jnp.float32),
              jax.ShapeDtypeStruct((E,), jnp.int32)],
    mesh=_mesh,
    compiler_params=pltpu.CompilerParams(needs_layout_passes=False),
    scratch_types=[
        pltpu.VMEM((EPW,), jnp.int32),
        pltpu.VMEM((EPW,), jnp.int32),
        pltpu.VMEM((EPW,), jnp.int32),
        pltpu.VMEM((EPW,), jnp.int32),
        pltpu.VMEM((SUBC,), jnp.int32),
        pltpu.VMEM((SUBC,), jnp.int32),
        pltpu.VMEM((TAIL,), jnp.int32),
        pltpu.VMEM((SUBC,), jnp.float32),
        pltpu.VMEM((CNT_SL,), jnp.float32),
        pltpu.SemaphoreType.DMA,
        pltpu.SemaphoreType.DMA,
        pltpu.VMEM_SHARED((NPAD,), jnp.float32),
    ],
)(_cnt_body)


def _inv_body(cnt_ref, inv_ref):
    cb = cnt_ref[...]
    inv_ref[...] = 1.0 / jnp.maximum(cb[0] + cb[1], 1.0)


def _inv_call(cnt3):
    nr = NPAD // D
    return pl.pallas_call(
        _inv_body,
        grid=(1,),
        in_specs=[pl.BlockSpec((NC, nr, D), lambda i: (0, 0, 0))],
        out_specs=pl.BlockSpec((nr, D), lambda i: (0, 0)),
        out_shape=jax.ShapeDtypeStruct((nr, D), jnp.float32),
    )(cnt3)


NITER = NFULL // 2


def _main_body(y_hbm, ep_hbm, inv_hbm, rb_hbm, out0_hbm, out1_hbm,
               pbuf, rows0, rows1, scale0, scale1,
               gidx0, didx0, sidx0, gidx1, didx1, sidx1,
               gidxt, didxt, sidxt,
               semg0, semg1, sems0, sems1, acc):
    c = lax.axis_index("c")
    s = lax.axis_index("s")
    wid = _wid()

    pltpu.sync_copy(ep_hbm.at[pl.ds(wid * EPW, EPW)], pbuf)

    z0 = s * ZROWS
    rem = ZROWS - (ZROWS // SUBC) * SUBC

    @pl.when(c == 0)
    def _():
        for q in range(ZROWS // SUBC):
            pltpu.sync_copy(rb_hbm.at[pl.ds(z0 + q * SUBC, SUBC)],
                            acc.at[pl.ds(z0 + q * SUBC, SUBC)])
        pltpu.sync_copy(rb_hbm.at[pl.ds(z0 + (ZROWS // SUBC) * SUBC, rem)],
                        acc.at[pl.ds(z0 + (ZROWS // SUBC) * SUBC, rem)])

        @pl.when(s == NS - 1)
        def _():
            pltpu.sync_copy(rb_hbm.at[pl.ds(NS * ZROWS, N - NS * ZROWS)],
                            acc.at[pl.ds(NS * ZROWS, N - NS * ZROWS)])

    @pl.when(c == 1)
    def _():
        def zrow(r, _):
            for cc in range(D // 16):
                rows0[r, pl.ds(cc * 16, 16)] = jnp.zeros((16,), jnp.float32)
            return 0
        lax.fori_loop(0, SUBC, zrow, 0)
        for q in range(ZROWS // SUBC):
            pltpu.sync_copy(rows0, acc.at[pl.ds(z0 + q * SUBC, SUBC)])
        pltpu.sync_copy(rows0.at[pl.ds(0, rem)],
                        acc.at[pl.ds(z0 + (ZROWS // SUBC) * SUBC, rem)])

        @pl.when(s == NS - 1)
        def _():
            pltpu.sync_copy(rows0.at[pl.ds(0, N - NS * ZROWS)],
                            acc.at[pl.ds(NS * ZROWS, N - NS * ZROWS)])
    plsc.subcore_barrier()

    def prep(j, gi, di, si, rw, sc, sg):
        base = j * SUBC
        for m in range(SUBC // 16):
            p = pbuf[pl.ds(base + m * 16, 16)]
            sv, dv, tv = _unpack(p)
            gi[pl.ds(m * 16, 16)] = tv * N + sv
            di[pl.ds(m * 16, 16)] = dv
            si[pl.ds(m * 16, 16)] = dv * NREL + tv
        pltpu.async_copy(y_hbm.at[gi], rw, sg)
        pltpu.async_copy(inv_hbm.at[si], sc, sg)

    def proc(gi, di, si, rw, sc, sg, ss):
        pltpu.make_async_copy(y_hbm.at[gi], rw, sg).wait()
        pltpu.make_async_copy(inv_hbm.at[si], sc, sg).wait()

        def mrow(r, _):
            s16 = plsc.load_gather(sc, [jnp.full((16,), r, jnp.int32)])
            for cc in range(D // 16):
                rw[r, pl.ds(cc * 16, 16)] = rw[r, pl.ds(cc * 16, 16)] * s16
            return 0
        lax.fori_loop(0, SUBC, mrow, 0, unroll=4)
        pltpu.async_copy(rw, acc.at[di], ss, add=True)

    def drain(rw, di, ss):
        pltpu.make_async_copy(rw, acc.at[di], ss).wait()

    set0 = (gidx0, didx0, sidx0, rows0, scale0, semg0)
    set1 = (gidx1, didx1, sidx1, rows1, scale1, semg1)

    prep(0, *set0)
    prep(1, *set1)

    def body(i, _):
        j0 = 2 * i
        proc(*set0[:5], semg0, sems0)
        proc(*set1[:5], semg1, sems1)

        @pl.when(i < NITER - 1)
        def _():
            drain(rows0, didx0, sems0)
            prep(j0 + 2, *set0)
            drain(rows1, didx1, sems1)
            prep(j0 + 3, *set1)
        return 0
    lax.fori_loop(0, NITER, body, 0)
    drain(rows0, didx0, sems0)
    drain(rows1, didx1, sems1)

    p = pbuf[pl.ds(NFULL * SUBC, TAIL)]
    sv, dv, tv = _unpack(p)
    gidxt[...] = tv * N + sv
    didxt[...] = dv
    sidxt[...] = dv * NREL + tv
    cp1 = pltpu.async_copy(y_hbm.at[gidxt], rows0.at[pl.ds(0, TAIL)], semg0)
    cp2 = pltpu.async_copy(inv_hbm.at[sidxt], scale0.at[pl.ds(0, TAIL)], semg0)
    cp1.wait()
    cp2.wait()

    def mrowt(r, _):
        s16 = plsc.load_gather(scale0, [jnp.full((16,), r, jnp.int32)])
        for cc in range(D // 16):
            rows0[r, pl.ds(cc * 16, 16)] = rows0[r, pl.ds(cc * 16, 16)] * s16
        return 0
    lax.fori_loop(0, TAIL, mrowt, 0)
    pltpu.sync_copy(rows0.at[pl.ds(0, TAIL)], acc.at[didxt], add=True)

    plsc.subcore_barrier()
    r0 = s * ZROWS
    for cc_, o_hbm in ((0, out0_hbm), (1, out1_hbm)):
        @pl.when(c == cc_)
        def _(o_hbm=o_hbm):
            for q in range(ZROWS // SUBC):
                pltpu.sync_copy(acc.at[pl.ds(r0 + q * SUBC, SUBC)],
                                o_hbm.at[pl.ds(r0 + q * SUBC, SUBC)])
            pltpu.sync_copy(acc.at[pl.ds(r0 + (ZROWS // SUBC) * SUBC, rem)],
                            o_hbm.at[pl.ds(r0 + (ZROWS // SUBC) * SUBC, rem)])

            @pl.when(s == NS - 1)
            def _():
                pltpu.sync_copy(acc.at[pl.ds(NS * ZROWS, N - NS * ZROWS)],
                                o_hbm.at[pl.ds(NS * ZROWS, N - NS * ZROWS)])


_main_call = functools.partial(
    pl.kernel,
    out_type=[jax.ShapeDtypeStruct((N, D), jnp.float32),
              jax.ShapeDtypeStruct((N, D), jnp.float32)],
    mesh=_mesh,
    compiler_params=pltpu.CompilerParams(needs_layout_passes=False),
    scratch_types=[
        pltpu.VMEM((EPW,), jnp.int32),
        pltpu.VMEM((SUBC, D), jnp.float32),
        pltpu.VMEM((SUBC, D), jnp.float32),
        pltpu.VMEM((SUBC,), jnp.float32),
        pltpu.VMEM((SUBC,), jnp.float32),
        pltpu.VMEM((SUBC,), jnp.int32),
        pltpu.VMEM((SUBC,), jnp.int32),
        pltpu.VMEM((SUBC,), jnp.int32),
        pltpu.VMEM((SUBC,), jnp.int32),
        pltpu.VMEM((SUBC,), jnp.int32),
        pltpu.VMEM((SUBC,), jnp.int32),
        pltpu.VMEM((TAIL,), jnp.int32),
        pltpu.VMEM((TAIL,), jnp.int32),
        pltpu.VMEM((TAIL,), jnp.int32),
        pltpu.SemaphoreType.DMA,
        pltpu.SemaphoreType.DMA,
        pltpu.SemaphoreType.DMA,
        pltpu.SemaphoreType.DMA,
        pltpu.VMEM_SHARED((N, D), jnp.float32),
    ],
)(_main_body)


_EPT = B // NW


def _emb_body(m0_hbm, m1_hbm, nest_hbm, food_hbm,
              ne_hbm, fe_hbm, idxn, idxf, m0n, m1n, m0f, m1f, semn, semf):
    wid = _wid()
    base = wid * _EPT
    pltpu.sync_copy(nest_hbm.at[pl.ds(base, _EPT)], idxn)
    pltpu.sync_copy(food_hbm.at[pl.ds(base, _EPT)], idxf)
    pltpu.async_copy(m0_hbm.at[idxn], m0n, semn)
    pltpu.async_copy(m1_hbm.at[idxn], m1n, semn)
    pltpu.async_copy(m0_hbm.at[idxf], m0f, semf)
    pltpu.async_copy(m1_hbm.at[idxf], m1f, semf)

    def combine(m0b, m1b):
        def row(r, _):
            for cc in range(D // 16):
                ds = pl.ds(cc * 16, 16)
                m0b[r, ds] = jnp.maximum(m0b[r, ds] + m1b[r, ds], 0.0)
            return 0
        lax.fori_loop(0, _EPT, row, 0, unroll=4)

    pltpu.make_async_copy(m0_hbm.at[idxn], m0n, semn).wait()
    pltpu.make_async_copy(m1_hbm.at[idxn], m1n, semn).wait()
    combine(m0n, m1n)
    pltpu.async_copy(m0n, ne_hbm.at[pl.ds(base, _EPT)], semn)

    pltpu.make_async_copy(m0_hbm.at[idxf], m0f, semf).wait()
    pltpu.make_async_copy(m1_hbm.at[idxf], m1f, semf).wait()
    combine(m0f, m1f)
    pltpu.async_copy(m0f, fe_hbm.at[pl.ds(base, _EPT)], semf)

    pltpu.make_async_copy(m0n, ne_hbm.at[pl.ds(base, _EPT)], semn).wait()
    pltpu.make_async_copy(m0f, fe_hbm.at[pl.ds(base, _EPT)], semf).wait()


_emb_call = functools.partial(
    pl.kernel,
    out_type=[jax.ShapeDtypeStruct((B, D), jnp.float32),
              jax.ShapeDtypeStruct((B, D), jnp.float32)],
    mesh=_mesh,
    compiler_params=pltpu.CompilerParams(needs_layout_passes=False),
    scratch_types=[
        pltpu.VMEM((_EPT,), jnp.int32),
        pltpu.VMEM((_EPT,), jnp.int32),
        pltpu.VMEM((_EPT, D), jnp.float32),
        pltpu.VMEM((_EPT, D), jnp.float32),
        pltpu.VMEM((_EPT, D), jnp.float32),
        pltpu.VMEM((_EPT, D), jnp.float32),
        pltpu.SemaphoreType.DMA,
        pltpu.SemaphoreType.DMA,
    ],
)(_emb_body)


def _head_body(ne_ref, fe_ref, wfc_ref, bfc_ref, wdir_ref, bdir_ref,
               wdist_ref, bdist_ref, la_ref, tb_ref):
    hid = jnp.dot(ne_ref[...], wfc_ref[:D], preferred_element_type=jnp.float32)
    hid = hid + jnp.dot(fe_ref[...], wfc_ref[D:],
                        preferred_element_type=jnp.float32)
    hid = jnp.maximum(hid + bfc_ref[...], 0.0)
    logit = jnp.dot(hid, wdir_ref[...], preferred_element_type=jnp.float32)
    logit = logit + bdir_ref[...]
    m = jnp.max(logit, axis=-1, keepdims=True)
    lse = jnp.log(jnp.sum(jnp.exp(logit - m), axis=-1, keepdims=True)) + m
    la_ref[...] = logit - lse
    tb_ref[...] = (jnp.dot(hid, wdist_ref[...],
                           preferred_element_type=jnp.float32)
                   + bdist_ref[...])


def _head_call(ne, fe, W_fc, b_fc, W_dir, b_dir, W_dist, b_dist):
    bm = 512
    grid = (B // bm,)
    return pl.pallas_call(
        _head_body,
        grid=grid,
        in_specs=[
            pl.BlockSpec((bm, D), lambda i: (i, 0)),
            pl.BlockSpec((bm, D), lambda i: (i, 0)),
            pl.BlockSpec((2 * D, HIDDEN), lambda i: (0, 0)),
            pl.BlockSpec((1, HIDDEN), lambda i: (0, 0)),
            pl.BlockSpec((HIDDEN, VOCAB), lambda i: (0, 0)),
            pl.BlockSpec((1, VOCAB), lambda i: (0, 0)),
            pl.BlockSpec((HIDDEN, 1), lambda i: (0, 0)),
            pl.BlockSpec((1, 1), lambda i: (0, 0)),
        ],
        out_specs=[
            pl.BlockSpec((bm, VOCAB), lambda i: (i, 0)),
            pl.BlockSpec((bm, 1), lambda i: (i, 0)),
        ],
        out_shape=[
            jax.ShapeDtypeStruct((B, VOCAB), jnp.float32),
            jax.ShapeDtypeStruct((B, 1), jnp.float32),
        ],
    )(ne, fe, W_fc, b_fc, W_dir, b_dir, W_dist, b_dist)


def kernel(x, edge_index, edge_type, nest, food, W_rel, W_root, b_rgcn,
           W_fc, b_fc, W_dir, b_dir, W_dist, b_dist):
    src = edge_index[0].astype(jnp.int32)
    dst = edge_index[1].astype(jnp.int32)
    et = edge_type.astype(jnp.int32)
    nest32 = nest.astype(jnp.int32)
    food32 = food.astype(jnp.int32)

    Y, rootb = _mm_call(x, W_rel, W_root, b_rgcn.reshape(1, -1))
    Y2 = Y.reshape(NREL * N, D)
    cnt2, epack = _cnt_call(src, dst, et)
    inv = _inv_call(cnt2.reshape(NC, NPAD // D, D)).reshape(NPAD)
    msg0, msg1 = _main_call(Y2, epack, inv, rootb)
    ne, fe = _emb_call(msg0, msg1, nest32, food32)
    la, tb = _head_call(ne, fe, W_fc, b_fc.reshape(1, -1),
                        W_dir, b_dir.reshape(1, -1),
                        W_dist, b_dist.reshape(1, -1))
    return (la, tb)


# speedup vs baseline: 19.9214x; 1.0009x over previous
import functools

import jax
import jax.numpy as jnp
from jax import lax
from jax.experimental import pallas as pl
from jax.experimental.pallas import tpu as pltpu
from jax.experimental.pallas import tpu_sc as plsc

N = 10000
E = 320000
D = 128
NREL = 4
B = 4096
HIDDEN = 256
VOCAB = 8

NC = 2
NS = 16
NW = NC * NS
EPW = E // NW
SUB = 80
GRP = 5
BLK = SUB * GRP
NBLK = EPW // BLK
NPAD = 40960
ZROWS = 624
CNT_SL = NPAD // NS

_mesh = plsc.VectorSubcoreMesh(core_axis_name="c", subcore_axis_name="s")


def _wid():
    return lax.axis_index("s") * NC + lax.axis_index("c")


def _mm_body(x_ref, wrel_ref, wroot_ref, b_ref, y_ref, root_ref):
    xb = x_ref[...]
    for r in range(NREL):
        y_ref[r] = jnp.dot(xb, wrel_ref[r], preferred_element_type=jnp.float32)
    root_ref[...] = (jnp.dot(xb, wroot_ref[...],
                             preferred_element_type=jnp.float32)
                     + b_ref[...])


def _mm_call(x, W_rel, W_root, b):
    bm = 400
    grid = (N // bm,)
    return pl.pallas_call(
        _mm_body,
        grid=grid,
        in_specs=[
            pl.BlockSpec((bm, D), lambda i: (i, 0)),
            pl.BlockSpec((NREL, D, D), lambda i: (0, 0, 0)),
            pl.BlockSpec((D, D), lambda i: (0, 0)),
            pl.BlockSpec((1, D), lambda i: (0, 0)),
        ],
        out_specs=[
            pl.BlockSpec((NREL, bm, D), lambda i: (0, i, 0)),
            pl.BlockSpec((bm, D), lambda i: (i, 0)),
        ],
        out_shape=[
            jax.ShapeDtypeStruct((NREL, N, D), jnp.float32),
            jax.ShapeDtypeStruct((N, D), jnp.float32),
        ],
    )(x, W_rel, W_root, b)


SUBC = 128
NFULL = EPW // SUBC
TAIL = EPW - NFULL * SUBC
MASK14 = 16383


def _unpack(p):
    sv = p & MASK14
    dv = (p >> 14) & MASK14
    tv = p >> 28
    return sv, dv, tv


def _cnt_body(sb_hbm, db_hbm, tb_hbm, cnt_hbm, ep_hbm,
              sbuf, dbuf, tbuf, ebuf, sidx, sidx2, sidxt, ones, zbuf,
              semb0, semb1, acc):
    c = lax.axis_index("c")
    s = lax.axis_index("s")
    wid = _wid()

    pltpu.sync_copy(sb_hbm.at[pl.ds(wid * EPW, EPW)], sbuf)
    pltpu.sync_copy(db_hbm.at[pl.ds(wid * EPW, EPW)], dbuf)
    pltpu.sync_copy(tb_hbm.at[pl.ds(wid * EPW, EPW)], tbuf)

    def pk(i, _):
        sv = sbuf[pl.ds(i * 16, 16)]
        dv = dbuf[pl.ds(i * 16, 16)]
        tv = tbuf[pl.ds(i * 16, 16)]
        ebuf[pl.ds(i * 16, 16)] = sv + dv * 16384 + tv * 268435456
        return 0
    lax.fori_loop(0, EPW // 16, pk, 0, unroll=4)
    pltpu.sync_copy(ebuf, ep_hbm.at[pl.ds(wid * EPW, EPW)])

    for k in range(SUBC // 16):
        ones[pl.ds(k * 16, 16)] = jnp.full((16,), 1.0, jnp.float32)

    def zb(i, _):
        zbuf[pl.ds(i * 16, 16)] = jnp.zeros((16,), jnp.float32)
        return 0
    lax.fori_loop(0, CNT_SL // 16, zb, 0)
    pltpu.sync_copy(zbuf, acc.at[pl.ds(s * CNT_SL, CNT_SL)])
    plsc.subcore_barrier()

    def sget(j, si):
        base = j * SUBC
        for m in range(SUBC // 16):
            dv = dbuf[pl.ds(base + m * 16, 16)]
            tv = tbuf[pl.ds(base + m * 16, 16)]
            si[pl.ds(m * 16, 16)] = dv * NREL + tv

    sget(0, sidx)
    pltpu.async_copy(ones, acc.at[sidx], semb0, add=True)
    sget(1, sidx2)
    pltpu.async_copy(ones, acc.at[sidx2], semb1, add=True)

    def body(i, _):
        j0 = 2 * i
        pltpu.make_async_copy(ones, acc.at[sidx], semb0).wait()
        sget(j0 + 2, sidx)
        pltpu.async_copy(ones, acc.at[sidx], semb0, add=True)
        pltpu.make_async_copy(ones, acc.at[sidx2], semb1).wait()

        @pl.when(i < NFULL // 2 - 2)
        def _():
            sget(j0 + 3, sidx2)
            pltpu.async_copy(ones, acc.at[sidx2], semb1, add=True)
        return 0
    lax.fori_loop(0, NFULL // 2 - 1, body, 0)
    pltpu.make_async_copy(ones, acc.at[sidx], semb0).wait()
    sget(NFULL - 1, sidx2)
    pltpu.sync_copy(ones, acc.at[sidx2], add=True)
    dv = dbuf[pl.ds(NFULL * SUBC, TAIL)]
    tv = tbuf[pl.ds(NFULL * SUBC, TAIL)]
    sidxt[...] = dv * NREL + tv
    pltpu.sync_copy(ones.at[pl.ds(0, TAIL)], acc.at[sidxt], add=True)

    plsc.subcore_barrier()
    pltpu.sync_copy(acc.at[pl.ds(s * CNT_SL, CNT_SL)],
                    cnt_hbm.at[c, pl.ds(s * CNT_SL, CNT_SL)])


_cnt_call = functools.partial(
    pl.kernel,
    out_type=[jax.ShapeDtypeStruct((NC, NPAD), jnp.float32),
              jax.ShapeDtypeStruct((E,), jnp.int32)],
    mesh=_mesh,
    compiler_params=pltpu.CompilerParams(needs_layout_passes=False),
    scratch_types=[
        pltpu.VMEM((EPW,), jnp.int32),
        pltpu.VMEM((EPW,), jnp.int32),
        pltpu.VMEM((EPW,), jnp.int32),
        pltpu.VMEM((EPW,), jnp.int32),
        pltpu.VMEM((SUBC,), jnp.int32),
        pltpu.VMEM((SUBC,), jnp.int32),
        pltpu.VMEM((TAIL,), jnp.int32),
        pltpu.VMEM((SUBC,), jnp.float32),
        pltpu.VMEM((CNT_SL,), jnp.float32),
        pltpu.SemaphoreType.DMA,
        pltpu.SemaphoreType.DMA,
        pltpu.VMEM_SHARED((NPAD,), jnp.float32),
    ],
)(_cnt_body)


def _inv_body(cnt_ref, inv_ref):
    cb = cnt_ref[...]
    inv_ref[...] = 1.0 / jnp.maximum(cb[0] + cb[1], 1.0)


def _inv_call(cnt3):
    nr = NPAD // D
    return pl.pallas_call(
        _inv_body,
        grid=(1,),
        in_specs=[pl.BlockSpec((NC, nr, D), lambda i: (0, 0, 0))],
        out_specs=pl.BlockSpec((nr, D), lambda i: (0, 0)),
        out_shape=jax.ShapeDtypeStruct((nr, D), jnp.float32),
    )(cnt3)


NITER = NFULL // 2


def _main_body(y_hbm, ep_hbm, inv_hbm, rb_hbm, out0_hbm, out1_hbm,
               pbuf, rows0, rows1, scale0, scale1,
               gidx0, didx0, sidx0, gidx1, didx1, sidx1,
               gidxt, didxt, sidxt,
               semg0, semg1, sems0, sems1, acc):
    c = lax.axis_index("c")
    s = lax.axis_index("s")
    wid = _wid()

    pltpu.sync_copy(ep_hbm.at[pl.ds(wid * EPW, EPW)], pbuf)

    z0 = s * ZROWS
    rem = ZROWS - (ZROWS // SUBC) * SUBC

    @pl.when(c == 0)
    def _():
        for q in range(ZROWS // SUBC):
            pltpu.sync_copy(rb_hbm.at[pl.ds(z0 + q * SUBC, SUBC)],
                            acc.at[pl.ds(z0 + q * SUBC, SUBC)])
        pltpu.sync_copy(rb_hbm.at[pl.ds(z0 + (ZROWS // SUBC) * SUBC, rem)],
                        acc.at[pl.ds(z0 + (ZROWS // SUBC) * SUBC, rem)])

        @pl.when(s == NS - 1)
        def _():
            pltpu.sync_copy(rb_hbm.at[pl.ds(NS * ZROWS, N - NS * ZROWS)],
                            acc.at[pl.ds(NS * ZROWS, N - NS * ZROWS)])

    @pl.when(c == 1)
    def _():
        def zrow(r, _):
            for cc in range(D // 16):
                rows0[r, pl.ds(cc * 16, 16)] = jnp.zeros((16,), jnp.float32)
            return 0
        lax.fori_loop(0, SUBC, zrow, 0)
        for q in range(ZROWS // SUBC):
            pltpu.sync_copy(rows0, acc.at[pl.ds(z0 + q * SUBC, SUBC)])
        pltpu.sync_copy(rows0.at[pl.ds(0, rem)],
                        acc.at[pl.ds(z0 + (ZROWS // SUBC) * SUBC, rem)])

        @pl.when(s == NS - 1)
        def _():
            pltpu.sync_copy(rows0.at[pl.ds(0, N - NS * ZROWS)],
                            acc.at[pl.ds(NS * ZROWS, N - NS * ZROWS)])
    plsc.subcore_barrier()

    def prep(j, gi, di, si, rw, sc, sg):
        base = j * SUBC
        for m in range(SUBC // 16):
            p = pbuf[pl.ds(base + m * 16, 16)]
            sv, dv, tv = _unpack(p)
            gi[pl.ds(m * 16, 16)] = tv * N + sv
            di[pl.ds(m * 16, 16)] = dv
            si[pl.ds(m * 16, 16)] = dv * NREL + tv
        pltpu.async_copy(y_hbm.at[gi], rw, sg)
        pltpu.async_copy(inv_hbm.at[si], sc, sg)

    def proc(gi, di, si, rw, sc, sg, ss):
        pltpu.make_async_copy(y_hbm.at[gi], rw, sg).wait()
        pltpu.make_async_copy(inv_hbm.at[si], sc, sg).wait()

        def mrow(r, _):
            s16 = plsc.load_gather(sc, [jnp.full((16,), r, jnp.int32)])
            for cc in range(D // 16):
                rw[r, pl.ds(cc * 16, 16)] = rw[r, pl.ds(cc * 16, 16)] * s16
            return 0
        lax.fori_loop(0, SUBC, mrow, 0, unroll=8)
        pltpu.async_copy(rw, acc.at[di], ss, add=True)

    def drain(rw, di, ss):
        pltpu.make_async_copy(rw, acc.at[di], ss).wait()

    set0 = (gidx0, didx0, sidx0, rows0, scale0, semg0)
    set1 = (gidx1, didx1, sidx1, rows1, scale1, semg1)

    prep(0, *set0)
    prep(1, *set1)

    def body(i, _):
        j0 = 2 * i
        proc(*set0[:5], semg0, sems0)
        proc(*set1[:5], semg1, sems1)

        @pl.when(i < NITER - 1)
        def _():
            drain(rows0, didx0, sems0)
            prep(j0 + 2, *set0)
            drain(rows1, didx1, sems1)
            prep(j0 + 3, *set1)
        return 0
    lax.fori_loop(0, NITER, body, 0)
    drain(rows0, didx0, sems0)
    drain(rows1, didx1, sems1)

    p = pbuf[pl.ds(NFULL * SUBC, TAIL)]
    sv, dv, tv = _unpack(p)
    gidxt[...] = tv * N + sv
    didxt[...] = dv
    sidxt[...] = dv * NREL + tv
    cp1 = pltpu.async_copy(y_hbm.at[gidxt], rows0.at[pl.ds(0, TAIL)], semg0)
    cp2 = pltpu.async_copy(inv_hbm.at[sidxt], scale0.at[pl.ds(0, TAIL)], semg0)
    cp1.wait()
    cp2.wait()

    def mrowt(r, _):
        s16 = plsc.load_gather(scale0, [jnp.full((16,), r, jnp.int32)])
        for cc in range(D // 16):
            rows0[r, pl.ds(cc * 16, 16)] = rows0[r, pl.ds(cc * 16, 16)] * s16
        return 0
    lax.fori_loop(0, TAIL, mrowt, 0)
    pltpu.sync_copy(rows0.at[pl.ds(0, TAIL)], acc.at[didxt], add=True)

    plsc.subcore_barrier()
    r0 = s * ZROWS
    for cc_, o_hbm in ((0, out0_hbm), (1, out1_hbm)):
        @pl.when(c == cc_)
        def _(o_hbm=o_hbm):
            for q in range(ZROWS // SUBC):
                pltpu.sync_copy(acc.at[pl.ds(r0 + q * SUBC, SUBC)],
                                o_hbm.at[pl.ds(r0 + q * SUBC, SUBC)])
            pltpu.sync_copy(acc.at[pl.ds(r0 + (ZROWS // SUBC) * SUBC, rem)],
                            o_hbm.at[pl.ds(r0 + (ZROWS // SUBC) * SUBC, rem)])

            @pl.when(s == NS - 1)
            def _():
                pltpu.sync_copy(acc.at[pl.ds(NS * ZROWS, N - NS * ZROWS)],
                                o_hbm.at[pl.ds(NS * ZROWS, N - NS * ZROWS)])


_main_call = functools.partial(
    pl.kernel,
    out_type=[jax.ShapeDtypeStruct((N, D), jnp.float32),
              jax.ShapeDtypeStruct((N, D), jnp.float32)],
    mesh=_mesh,
    compiler_params=pltpu.CompilerParams(needs_layout_passes=False),
    scratch_types=[
        pltpu.VMEM((EPW,), jnp.int32),
        pltpu.VMEM((SUBC, D), jnp.float32),
        pltpu.VMEM((SUBC, D), jnp.float32),
        pltpu.VMEM((SUBC,), jnp.float32),
        pltpu.VMEM((SUBC,), jnp.float32),
        pltpu.VMEM((SUBC,), jnp.int32),
        pltpu.VMEM((SUBC,), jnp.int32),
        pltpu.VMEM((SUBC,), jnp.int32),
        pltpu.VMEM((SUBC,), jnp.int32),
        pltpu.VMEM((SUBC,), jnp.int32),
        pltpu.VMEM((SUBC,), jnp.int32),
        pltpu.VMEM((TAIL,), jnp.int32),
        pltpu.VMEM((TAIL,), jnp.int32),
        pltpu.VMEM((TAIL,), jnp.int32),
        pltpu.SemaphoreType.DMA,
        pltpu.SemaphoreType.DMA,
        pltpu.SemaphoreType.DMA,
        pltpu.SemaphoreType.DMA,
        pltpu.VMEM_SHARED((N, D), jnp.float32),
    ],
)(_main_body)


_EPT = B // NW


def _emb_body(m0_hbm, m1_hbm, nest_hbm, food_hbm,
              ne_hbm, fe_hbm, idxn, idxf, m0n, m1n, m0f, m1f, semn, semf):
    wid = _wid()
    base = wid * _EPT
    pltpu.sync_copy(nest_hbm.at[pl.ds(base, _EPT)], idxn)
    pltpu.sync_copy(food_hbm.at[pl.ds(base, _EPT)], idxf)
    pltpu.async_copy(m0_hbm.at[idxn], m0n, semn)
    pltpu.async_copy(m1_hbm.at[idxn], m1n, semn)
    pltpu.async_copy(m0_hbm.at[idxf], m0f, semf)
    pltpu.async_copy(m1_hbm.at[idxf], m1f, semf)

    def combine(m0b, m1b):
        def row(r, _):
            for cc in range(D // 16):
                ds = pl.ds(cc * 16, 16)
                m0b[r, ds] = jnp.maximum(m0b[r, ds] + m1b[r, ds], 0.0)
            return 0
        lax.fori_loop(0, _EPT, row, 0, unroll=4)

    pltpu.make_async_copy(m0_hbm.at[idxn], m0n, semn).wait()
    pltpu.make_async_copy(m1_hbm.at[idxn], m1n, semn).wait()
    combine(m0n, m1n)
    pltpu.async_copy(m0n, ne_hbm.at[pl.ds(base, _EPT)], semn)

    pltpu.make_async_copy(m0_hbm.at[idxf], m0f, semf).wait()
    pltpu.make_async_copy(m1_hbm.at[idxf], m1f, semf).wait()
    combine(m0f, m1f)
    pltpu.async_copy(m0f, fe_hbm.at[pl.ds(base, _EPT)], semf)

    pltpu.make_async_copy(m0n, ne_hbm.at[pl.ds(base, _EPT)], semn).wait()
    pltpu.make_async_copy(m0f, fe_hbm.at[pl.ds(base, _EPT)], semf).wait()


_emb_call = functools.partial(
    pl.kernel,
    out_type=[jax.ShapeDtypeStruct((B, D), jnp.float32),
              jax.ShapeDtypeStruct((B, D), jnp.float32)],
    mesh=_mesh,
    compiler_params=pltpu.CompilerParams(needs_layout_passes=False),
    scratch_types=[
        pltpu.VMEM((_EPT,), jnp.int32),
        pltpu.VMEM((_EPT,), jnp.int32),
        pltpu.VMEM((_EPT, D), jnp.float32),
        pltpu.VMEM((_EPT, D), jnp.float32),
        pltpu.VMEM((_EPT, D), jnp.float32),
        pltpu.VMEM((_EPT, D), jnp.float32),
        pltpu.SemaphoreType.DMA,
        pltpu.SemaphoreType.DMA,
    ],
)(_emb_body)


def _head_body(ne_ref, fe_ref, wfc_ref, bfc_ref, wdir_ref, bdir_ref,
               wdist_ref, bdist_ref, la_ref, tb_ref):
    hid = jnp.dot(ne_ref[...], wfc_ref[:D], preferred_element_type=jnp.float32)
    hid = hid + jnp.dot(fe_ref[...], wfc_ref[D:],
                        preferred_element_type=jnp.float32)
    hid = jnp.maximum(hid + bfc_ref[...], 0.0)
    logit = jnp.dot(hid, wdir_ref[...], preferred_element_type=jnp.float32)
    logit = logit + bdir_ref[...]
    m = jnp.max(logit, axis=-1, keepdims=True)
    lse = jnp.log(jnp.sum(jnp.exp(logit - m), axis=-1, keepdims=True)) + m
    la_ref[...] = logit - lse
    tb_ref[...] = (jnp.dot(hid, wdist_ref[...],
                           preferred_element_type=jnp.float32)
                   + bdist_ref[...])


def _head_call(ne, fe, W_fc, b_fc, W_dir, b_dir, W_dist, b_dist):
    bm = 512
    grid = (B // bm,)
    return pl.pallas_call(
        _head_body,
        grid=grid,
        in_specs=[
            pl.BlockSpec((bm, D), lambda i: (i, 0)),
            pl.BlockSpec((bm, D), lambda i: (i, 0)),
            pl.BlockSpec((2 * D, HIDDEN), lambda i: (0, 0)),
            pl.BlockSpec((1, HIDDEN), lambda i: (0, 0)),
            pl.BlockSpec((HIDDEN, VOCAB), lambda i: (0, 0)),
            pl.BlockSpec((1, VOCAB), lambda i: (0, 0)),
            pl.BlockSpec((HIDDEN, 1), lambda i: (0, 0)),
            pl.BlockSpec((1, 1), lambda i: (0, 0)),
        ],
        out_specs=[
            pl.BlockSpec((bm, VOCAB), lambda i: (i, 0)),
            pl.BlockSpec((bm, 1), lambda i: (i, 0)),
        ],
        out_shape=[
            jax.ShapeDtypeStruct((B, VOCAB), jnp.float32),
            jax.ShapeDtypeStruct((B, 1), jnp.float32),
        ],
    )(ne, fe, W_fc, b_fc, W_dir, b_dir, W_dist, b_dist)


def kernel(x, edge_index, edge_type, nest, food, W_rel, W_root, b_rgcn,
           W_fc, b_fc, W_dir, b_dir, W_dist, b_dist):
    src = edge_index[0].astype(jnp.int32)
    dst = edge_index[1].astype(jnp.int32)
    et = edge_type.astype(jnp.int32)
    nest32 = nest.astype(jnp.int32)
    food32 = food.astype(jnp.int32)

    Y, rootb = _mm_call(x, W_rel, W_root, b_rgcn.reshape(1, -1))
    Y2 = Y.reshape(NREL * N, D)
    cnt2, epack = _cnt_call(src, dst, et)
    inv = _inv_call(cnt2.reshape(NC, NPAD // D, D)).reshape(NPAD)
    msg0, msg1 = _main_call(Y2, epack, inv, rootb)
    ne, fe = _emb_call(msg0, msg1, nest32, food32)
    la, tb = _head_call(ne, fe, W_fc, b_fc.reshape(1, -1),
                        W_dir, b_dir.reshape(1, -1),
                        W_dist, b_dist.reshape(1, -1))
    return (la, tb)


# speedup vs baseline: 19.9291x; 1.0004x over previous
import functools

import jax
import jax.numpy as jnp
from jax import lax
from jax.experimental import pallas as pl
from jax.experimental.pallas import tpu as pltpu
from jax.experimental.pallas import tpu_sc as plsc

N = 10000
E = 320000
D = 128
NREL = 4
B = 4096
HIDDEN = 256
VOCAB = 8

NC = 2
NS = 16
NW = NC * NS
EPW = E // NW
SUB = 80
GRP = 5
BLK = SUB * GRP
NBLK = EPW // BLK
NPAD = 40960
ZROWS = 624
CNT_SL = NPAD // NS

_mesh = plsc.VectorSubcoreMesh(core_axis_name="c", subcore_axis_name="s",
                               num_cores=NC, num_subcores=NS)


def _wid():
    return lax.axis_index("s") * NC + lax.axis_index("c")


def _mm_body(x_ref, wrel_ref, wroot_ref, b_ref, y_ref, root_ref):
    xb = x_ref[...]
    for r in range(NREL):
        y_ref[r] = jnp.dot(xb, wrel_ref[r], preferred_element_type=jnp.float32)
    root_ref[...] = (jnp.dot(xb, wroot_ref[...],
                             preferred_element_type=jnp.float32)
                     + b_ref[...])


def _mm_call(x, W_rel, W_root, b):
    bm = 400
    grid = (N // bm,)
    return pl.pallas_call(
        _mm_body,
        grid=grid,
        in_specs=[
            pl.BlockSpec((bm, D), lambda i: (i, 0)),
            pl.BlockSpec((NREL, D, D), lambda i: (0, 0, 0)),
            pl.BlockSpec((D, D), lambda i: (0, 0)),
            pl.BlockSpec((1, D), lambda i: (0, 0)),
        ],
        out_specs=[
            pl.BlockSpec((NREL, bm, D), lambda i: (0, i, 0)),
            pl.BlockSpec((bm, D), lambda i: (i, 0)),
        ],
        out_shape=[
            jax.ShapeDtypeStruct((NREL, N, D), jnp.float32),
            jax.ShapeDtypeStruct((N, D), jnp.float32),
        ],
    )(x, W_rel, W_root, b)


SUBC = 128
NFULL = EPW // SUBC
TAIL = EPW - NFULL * SUBC
MASK14 = 16383


def _unpack(p):
    sv = p & MASK14
    dv = (p >> 14) & MASK14
    tv = p >> 28
    return sv, dv, tv


def _cnt_body(sb_hbm, db_hbm, tb_hbm, cnt_hbm, ep_hbm,
              sbuf, dbuf, tbuf, ebuf, sidx, sidx2, sidxt, ones, zbuf,
              semb0, semb1, acc):
    c = lax.axis_index("c")
    s = lax.axis_index("s")
    wid = _wid()

    pltpu.sync_copy(sb_hbm.at[pl.ds(wid * EPW, EPW)], sbuf)
    pltpu.sync_copy(db_hbm.at[pl.ds(wid * EPW, EPW)], dbuf)
    pltpu.sync_copy(tb_hbm.at[pl.ds(wid * EPW, EPW)], tbuf)

    def pk(i, _):
        sv = sbuf[pl.ds(i * 16, 16)]
        dv = dbuf[pl.ds(i * 16, 16)]
        tv = tbuf[pl.ds(i * 16, 16)]
        ebuf[pl.ds(i * 16, 16)] = sv + dv * 16384 + tv * 268435456
        return 0
    lax.fori_loop(0, EPW // 16, pk, 0, unroll=4)
    pltpu.sync_copy(ebuf, ep_hbm.at[pl.ds(wid * EPW, EPW)])

    for k in range(SUBC // 16):
        ones[pl.ds(k * 16, 16)] = jnp.full((16,), 1.0, jnp.float32)

    def zb(i, _):
        zbuf[pl.ds(i * 16, 16)] = jnp.zeros((16,), jnp.float32)
        return 0
    lax.fori_loop(0, CNT_SL // 16, zb, 0)
    pltpu.sync_copy(zbuf, acc.at[pl.ds(s * CNT_SL, CNT_SL)])
    plsc.subcore_barrier()

    def sget(j, si):
        base = j * SUBC
        for m in range(SUBC // 16):
            dv = dbuf[pl.ds(base + m * 16, 16)]
            tv = tbuf[pl.ds(base + m * 16, 16)]
            si[pl.ds(m * 16, 16)] = dv * NREL + tv

    sget(0, sidx)
    pltpu.async_copy(ones, acc.at[sidx], semb0, add=True)
    sget(1, sidx2)
    pltpu.async_copy(ones, acc.at[sidx2], semb1, add=True)

    def body(i, _):
        j0 = 2 * i
        pltpu.make_async_copy(ones, acc.at[sidx], semb0).wait()
        sget(j0 + 2, sidx)
        pltpu.async_copy(ones, acc.at[sidx], semb0, add=True)
        pltpu.make_async_copy(ones, acc.at[sidx2], semb1).wait()

        @pl.when(i < NFULL // 2 - 2)
        def _():
            sget(j0 + 3, sidx2)
            pltpu.async_copy(ones, acc.at[sidx2], semb1, add=True)
        return 0
    lax.fori_loop(0, NFULL // 2 - 1, body, 0)
    pltpu.make_async_copy(ones, acc.at[sidx], semb0).wait()
    sget(NFULL - 1, sidx2)
    pltpu.sync_copy(ones, acc.at[sidx2], add=True)
    dv = dbuf[pl.ds(NFULL * SUBC, TAIL)]
    tv = tbuf[pl.ds(NFULL * SUBC, TAIL)]
    sidxt[...] = dv * NREL + tv
    pltpu.sync_copy(ones.at[pl.ds(0, TAIL)], acc.at[sidxt], add=True)

    plsc.subcore_barrier()
    pltpu.sync_copy(acc.at[pl.ds(s * CNT_SL, CNT_SL)],
                    cnt_hbm.at[c, pl.ds(s * CNT_SL, CNT_SL)])


_cnt_call = functools.partial(
    pl.kernel,
    out_type=[jax.ShapeDtypeStruct((NC, NPAD), jnp.float32),
              jax.ShapeDtypeStruct((E,), jnp.int32)],
    mesh=_mesh,
    compiler_params=pltpu.CompilerParams(needs_layout_passes=False),
    scratch_types=[
        pltpu.VMEM((EPW,), jnp.int32),
        pltpu.VMEM((EPW,), jnp.int32),
        pltpu.VMEM((EPW,), jnp.int32),
        pltpu.VMEM((EPW,), jnp.int32),
        pltpu.VMEM((SUBC,), jnp.int32),
        pltpu.VMEM((SUBC,), jnp.int32),
        pltpu.VMEM((TAIL,), jnp.int32),
        pltpu.VMEM((SUBC,), jnp.float32),
        pltpu.VMEM((CNT_SL,), jnp.float32),
        pltpu.SemaphoreType.DMA,
        pltpu.SemaphoreType.DMA,
        pltpu.VMEM_SHARED((NPAD,), jnp.float32),
    ],
)(_cnt_body)


def _inv_body(cnt_ref, inv_ref):
    cb = cnt_ref[...]
    inv_ref[...] = 1.0 / jnp.maximum(cb[0] + cb[1], 1.0)


def _inv_call(cnt3):
    nr = NPAD // D
    return pl.pallas_call(
        _inv_body,
        grid=(1,),
        in_specs=[pl.BlockSpec((NC, nr, D), lambda i: (0, 0, 0))],
        out_specs=pl.BlockSpec((nr, D), lambda i: (0, 0)),
        out_shape=jax.ShapeDtypeStruct((nr, D), jnp.float32),
    )(cnt3)


NITER = NFULL // 2


def _main_body(y_hbm, ep_hbm, inv_hbm, rb_hbm, out0_hbm, out1_hbm,
               pbuf, rows0, rows1, scale0, scale1,
               gidx0, didx0, sidx0, gidx1, didx1, sidx1,
               gidxt, didxt, sidxt,
               semg0, semg1, sems0, sems1, acc):
    c = lax.axis_index("c")
    s = lax.axis_index("s")
    wid = _wid()

    pltpu.sync_copy(ep_hbm.at[pl.ds(wid * EPW, EPW)], pbuf)

    z0 = s * ZROWS
    rem = ZROWS - (ZROWS // SUBC) * SUBC

    @pl.when(c == 0)
    def _():
        for q in range(ZROWS // SUBC):
            pltpu.sync_copy(rb_hbm.at[pl.ds(z0 + q * SUBC, SUBC)],
                            acc.at[pl.ds(z0 + q * SUBC, SUBC)])
        pltpu.sync_copy(rb_hbm.at[pl.ds(z0 + (ZROWS // SUBC) * SUBC, rem)],
                        acc.at[pl.ds(z0 + (ZROWS // SUBC) * SUBC, rem)])

        @pl.when(s == NS - 1)
        def _():
            pltpu.sync_copy(rb_hbm.at[pl.ds(NS * ZROWS, N - NS * ZROWS)],
                            acc.at[pl.ds(NS * ZROWS, N - NS * ZROWS)])

    @pl.when(c == 1)
    def _():
        def zrow(r, _):
            for cc in range(D // 16):
                rows0[r, pl.ds(cc * 16, 16)] = jnp.zeros((16,), jnp.float32)
            return 0
        lax.fori_loop(0, SUBC, zrow, 0)
        for q in range(ZROWS // SUBC):
            pltpu.sync_copy(rows0, acc.at[pl.ds(z0 + q * SUBC, SUBC)])
        pltpu.sync_copy(rows0.at[pl.ds(0, rem)],
                        acc.at[pl.ds(z0 + (ZROWS // SUBC) * SUBC, rem)])

        @pl.when(s == NS - 1)
        def _():
            pltpu.sync_copy(rows0.at[pl.ds(0, N - NS * ZROWS)],
                            acc.at[pl.ds(NS * ZROWS, N - NS * ZROWS)])
    plsc.subcore_barrier()

    def prep(j, gi, di, si, rw, sc, sg):
        base = j * SUBC
        for m in range(SUBC // 16):
            p = pbuf[pl.ds(base + m * 16, 16)]
            sv, dv, tv = _unpack(p)
            gi[pl.ds(m * 16, 16)] = tv * N + sv
            di[pl.ds(m * 16, 16)] = dv
            si[pl.ds(m * 16, 16)] = dv * NREL + tv
        pltpu.async_copy(y_hbm.at[gi], rw, sg)
        pltpu.async_copy(inv_hbm.at[si], sc, sg)

    def proc(gi, di, si, rw, sc, sg, ss):
        pltpu.make_async_copy(y_hbm.at[gi], rw, sg).wait()
        pltpu.make_async_copy(inv_hbm.at[si], sc, sg).wait()

        def mrow(r, _):
            s16 = plsc.load_gather(sc, [jnp.full((16,), r, jnp.int32)])
            for cc in range(D // 16):
                rw[r, pl.ds(cc * 16, 16)] = rw[r, pl.ds(cc * 16, 16)] * s16
            return 0
        lax.fori_loop(0, SUBC, mrow, 0, unroll=8)
        pltpu.async_copy(rw, acc.at[di], ss, add=True)

    def drain(rw, di, ss):
        pltpu.make_async_copy(rw, acc.at[di], ss).wait()

    set0 = (gidx0, didx0, sidx0, rows0, scale0, semg0)
    set1 = (gidx1, didx1, sidx1, rows1, scale1, semg1)

    prep(0, *set0)
    prep(1, *set1)

    def body(i, _):
        j0 = 2 * i
        proc(*set0[:5], semg0, sems0)
        proc(*set1[:5], semg1, sems1)

        @pl.when(i < NITER - 1)
        def _():
            drain(rows0, didx0, sems0)
            prep(j0 + 2, *set0)
            drain(rows1, didx1, sems1)
            prep(j0 + 3, *set1)
        return 0
    lax.fori_loop(0, NITER, body, 0)
    drain(rows0, didx0, sems0)
    drain(rows1, didx1, sems1)

    p = pbuf[pl.ds(NFULL * SUBC, TAIL)]
    sv, dv, tv = _unpack(p)
    gidxt[...] = tv * N + sv
    didxt[...] = dv
    sidxt[...] = dv * NREL + tv
    cp1 = pltpu.async_copy(y_hbm.at[gidxt], rows0.at[pl.ds(0, TAIL)], semg0)
    cp2 = pltpu.async_copy(inv_hbm.at[sidxt], scale0.at[pl.ds(0, TAIL)], semg0)
    cp1.wait()
    cp2.wait()

    def mrowt(r, _):
        s16 = plsc.load_gather(scale0, [jnp.full((16,), r, jnp.int32)])
        for cc in range(D // 16):
            rows0[r, pl.ds(cc * 16, 16)] = rows0[r, pl.ds(cc * 16, 16)] * s16
        return 0
    lax.fori_loop(0, TAIL, mrowt, 0)
    pltpu.sync_copy(rows0.at[pl.ds(0, TAIL)], acc.at[didxt], add=True)

    plsc.subcore_barrier()
    r0 = s * ZROWS
    for cc_, o_hbm in ((0, out0_hbm), (1, out1_hbm)):
        @pl.when(c == cc_)
        def _(o_hbm=o_hbm):
            for q in range(ZROWS // SUBC):
                pltpu.sync_copy(acc.at[pl.ds(r0 + q * SUBC, SUBC)],
                                o_hbm.at[pl.ds(r0 + q * SUBC, SUBC)])
            pltpu.sync_copy(acc.at[pl.ds(r0 + (ZROWS // SUBC) * SUBC, rem)],
                            o_hbm.at[pl.ds(r0 + (ZROWS // SUBC) * SUBC, rem)])

            @pl.when(s == NS - 1)
            def _():
                pltpu.sync_copy(acc.at[pl.ds(NS * ZROWS, N - NS * ZROWS)],
                                o_hbm.at[pl.ds(NS * ZROWS, N - NS * ZROWS)])


_main_call = functools.partial(
    pl.kernel,
    out_type=[jax.ShapeDtypeStruct((N, D), jnp.float32),
              jax.ShapeDtypeStruct((N, D), jnp.float32)],
    mesh=_mesh,
    compiler_params=pltpu.CompilerParams(needs_layout_passes=False),
    scratch_types=[
        pltpu.VMEM((EPW,), jnp.int32),
        pltpu.VMEM((SUBC, D), jnp.float32),
        pltpu.VMEM((SUBC, D), jnp.float32),
        pltpu.VMEM((SUBC,), jnp.float32),
        pltpu.VMEM((SUBC,), jnp.float32),
        pltpu.VMEM((SUBC,), jnp.int32),
        pltpu.VMEM((SUBC,), jnp.int32),
        pltpu.VMEM((SUBC,), jnp.int32),
        pltpu.VMEM((SUBC,), jnp.int32),
        pltpu.VMEM((SUBC,), jnp.int32),
        pltpu.VMEM((SUBC,), jnp.int32),
        pltpu.VMEM((TAIL,), jnp.int32),
        pltpu.VMEM((TAIL,), jnp.int32),
        pltpu.VMEM((TAIL,), jnp.int32),
        pltpu.SemaphoreType.DMA,
        pltpu.SemaphoreType.DMA,
        pltpu.SemaphoreType.DMA,
        pltpu.SemaphoreType.DMA,
        pltpu.VMEM_SHARED((N, D), jnp.float32),
    ],
)(_main_body)


_EPT = B // NW


def _emb_body(m0_hbm, m1_hbm, nest_hbm, food_hbm,
              ne_hbm, fe_hbm, idxn, idxf, m0n, m1n, m0f, m1f, semn, semf):
    wid = _wid()
    base = wid * _EPT
    pltpu.sync_copy(nest_hbm.at[pl.ds(base, _EPT)], idxn)
    pltpu.sync_copy(food_hbm.at[pl.ds(base, _EPT)], idxf)
    pltpu.async_copy(m0_hbm.at[idxn], m0n, semn)
    pltpu.async_copy(m1_hbm.at[idxn], m1n, semn)
    pltpu.async_copy(m0_hbm.at[idxf], m0f, semf)
    pltpu.async_copy(m1_hbm.at[idxf], m1f, semf)

    def combine(m0b, m1b):
        def row(r, _):
            for cc in range(D // 16):
                ds = pl.ds(cc * 16, 16)
                m0b[r, ds] = jnp.maximum(m0b[r, ds] + m1b[r, ds], 0.0)
            return 0
        lax.fori_loop(0, _EPT, row, 0, unroll=4)

    pltpu.make_async_copy(m0_hbm.at[idxn], m0n, semn).wait()
    pltpu.make_async_copy(m1_hbm.at[idxn], m1n, semn).wait()
    combine(m0n, m1n)
    pltpu.async_copy(m0n, ne_hbm.at[pl.ds(base, _EPT)], semn)

    pltpu.make_async_copy(m0_hbm.at[idxf], m0f, semf).wait()
    pltpu.make_async_copy(m1_hbm.at[idxf], m1f, semf).wait()
    combine(m0f, m1f)
    pltpu.async_copy(m0f, fe_hbm.at[pl.ds(base, _EPT)], semf)

    pltpu.make_async_copy(m0n, ne_hbm.at[pl.ds(base, _EPT)], semn).wait()
    pltpu.make_async_copy(m0f, fe_hbm.at[pl.ds(base, _EPT)], semf).wait()


_emb_call = functools.partial(
    pl.kernel,
    out_type=[jax.ShapeDtypeStruct((B, D), jnp.float32),
              jax.ShapeDtypeStruct((B, D), jnp.float32)],
    mesh=_mesh,
    compiler_params=pltpu.CompilerParams(needs_layout_passes=False),
    scratch_types=[
        pltpu.VMEM((_EPT,), jnp.int32),
        pltpu.VMEM((_EPT,), jnp.int32),
        pltpu.VMEM((_EPT, D), jnp.float32),
        pltpu.VMEM((_EPT, D), jnp.float32),
        pltpu.VMEM((_EPT, D), jnp.float32),
        pltpu.VMEM((_EPT, D), jnp.float32),
        pltpu.SemaphoreType.DMA,
        pltpu.SemaphoreType.DMA,
    ],
)(_emb_body)


def _head_body(ne_ref, fe_ref, wfc_ref, bfc_ref, wdir_ref, bdir_ref,
               wdist_ref, bdist_ref, la_ref, tb_ref):
    hid = jnp.dot(ne_ref[...], wfc_ref[:D], preferred_element_type=jnp.float32)
    hid = hid + jnp.dot(fe_ref[...], wfc_ref[D:],
                        preferred_element_type=jnp.float32)
    hid = jnp.maximum(hid + bfc_ref[...], 0.0)
    logit = jnp.dot(hid, wdir_ref[...], preferred_element_type=jnp.float32)
    logit = logit + bdir_ref[...]
    m = jnp.max(logit, axis=-1, keepdims=True)
    lse = jnp.log(jnp.sum(jnp.exp(logit - m), axis=-1, keepdims=True)) + m
    la_ref[...] = logit - lse
    tb_ref[...] = (jnp.dot(hid, wdist_ref[...],
                           preferred_element_type=jnp.float32)
                   + bdist_ref[...])


def _head_call(ne, fe, W_fc, b_fc, W_dir, b_dir, W_dist, b_dist):
    bm = 512
    grid = (B // bm,)
    return pl.pallas_call(
        _head_body,
        grid=grid,
        in_specs=[
            pl.BlockSpec((bm, D), lambda i: (i, 0)),
            pl.BlockSpec((bm, D), lambda i: (i, 0)),
            pl.BlockSpec((2 * D, HIDDEN), lambda i: (0, 0)),
            pl.BlockSpec((1, HIDDEN), lambda i: (0, 0)),
            pl.BlockSpec((HIDDEN, VOCAB), lambda i: (0, 0)),
            pl.BlockSpec((1, VOCAB), lambda i: (0, 0)),
            pl.BlockSpec((HIDDEN, 1), lambda i: (0, 0)),
            pl.BlockSpec((1, 1), lambda i: (0, 0)),
        ],
        out_specs=[
            pl.BlockSpec((bm, VOCAB), lambda i: (i, 0)),
            pl.BlockSpec((bm, 1), lambda i: (i, 0)),
        ],
        out_shape=[
            jax.ShapeDtypeStruct((B, VOCAB), jnp.float32),
            jax.ShapeDtypeStruct((B, 1), jnp.float32),
        ],
    )(ne, fe, W_fc, b_fc, W_dir, b_dir, W_dist, b_dist)


def kernel(x, edge_index, edge_type, nest, food, W_rel, W_root, b_rgcn,
           W_fc, b_fc, W_dir, b_dir, W_dist, b_dist):
    src = edge_index[0].astype(jnp.int32)
    dst = edge_index[1].astype(jnp.int32)
    et = edge_type.astype(jnp.int32)
    nest32 = nest.astype(jnp.int32)
    food32 = food.astype(jnp.int32)

    Y, rootb = _mm_call(x, W_rel, W_root, b_rgcn.reshape(1, -1))
    Y2 = Y.reshape(NREL * N, D)
    cnt2, epack = _cnt_call(src, dst, et)
    inv = _inv_call(cnt2.reshape(NC, NPAD // D, D)).reshape(NPAD)
    msg0, msg1 = _main_call(Y2, epack, inv, rootb)
    ne, fe = _emb_call(msg0, msg1, nest32, food32)
    la, tb = _head_call(ne, fe, W_fc, b_fc.reshape(1, -1),
                        W_dir, b_dir.reshape(1, -1),
                        W_dist, b_dist.reshape(1, -1))
    return (la, tb)


# speedup vs baseline: 20.0971x; 1.0084x over previous
import functools

import jax
import jax.numpy as jnp
from jax import lax
from jax.experimental import pallas as pl
from jax.experimental.pallas import tpu as pltpu
from jax.experimental.pallas import tpu_sc as plsc

N = 10000
E = 320000
D = 128
NREL = 4
B = 4096
HIDDEN = 256
VOCAB = 8

NC = 2
NS = 16
NW = NC * NS
EPW = E // NW
SUB = 80
GRP = 5
BLK = SUB * GRP
NBLK = EPW // BLK
NPAD = 40960
ZROWS = 624
CNT_SL = NPAD // NS

_mesh = plsc.VectorSubcoreMesh(core_axis_name="c", subcore_axis_name="s",
                               num_cores=NC, num_subcores=NS)


def _wid():
    return lax.axis_index("s") * NC + lax.axis_index("c")


def _mm_body(x_ref, wrel_ref, wroot_ref, b_ref, y_ref, root_ref):
    xb = x_ref[...]
    for r in range(NREL):
        y_ref[r] = jnp.dot(xb, wrel_ref[r], preferred_element_type=jnp.float32)
    root_ref[...] = (jnp.dot(xb, wroot_ref[...],
                             preferred_element_type=jnp.float32)
                     + b_ref[...])


def _mm_call(x, W_rel, W_root, b):
    bm = 400
    grid = (N // bm,)
    return pl.pallas_call(
        _mm_body,
        grid=grid,
        in_specs=[
            pl.BlockSpec((bm, D), lambda i: (i, 0)),
            pl.BlockSpec((NREL, D, D), lambda i: (0, 0, 0)),
            pl.BlockSpec((D, D), lambda i: (0, 0)),
            pl.BlockSpec((1, D), lambda i: (0, 0)),
        ],
        out_specs=[
            pl.BlockSpec((NREL, bm, D), lambda i: (0, i, 0)),
            pl.BlockSpec((bm, D), lambda i: (i, 0)),
        ],
        out_shape=[
            jax.ShapeDtypeStruct((NREL, N, D), jnp.float32),
            jax.ShapeDtypeStruct((N, D), jnp.float32),
        ],
    )(x, W_rel, W_root, b)


SUBC = 128
NFULL = EPW // SUBC
TAIL = EPW - NFULL * SUBC
MASK14 = 16383


def _unpack(p):
    sv = p & MASK14
    dv = (p >> 14) & MASK14
    tv = p >> 28
    return sv, dv, tv


def _cnt_body(sb_hbm, db_hbm, tb_hbm, cnt_hbm, ep_hbm,
              sbuf, dbuf, tbuf, ebuf, sidx, sidx2, sidxt, ones, zbuf,
              semb0, semb1, acc):
    c = lax.axis_index("c")
    s = lax.axis_index("s")
    wid = _wid()

    pltpu.sync_copy(sb_hbm.at[pl.ds(wid * EPW, EPW)], sbuf)
    pltpu.sync_copy(db_hbm.at[pl.ds(wid * EPW, EPW)], dbuf)
    pltpu.sync_copy(tb_hbm.at[pl.ds(wid * EPW, EPW)], tbuf)

    def pk(i, _):
        sv = sbuf[pl.ds(i * 16, 16)]
        dv = dbuf[pl.ds(i * 16, 16)]
        tv = tbuf[pl.ds(i * 16, 16)]
        ebuf[pl.ds(i * 16, 16)] = sv + dv * 16384 + tv * 268435456
        return 0
    lax.fori_loop(0, EPW // 16, pk, 0, unroll=4)
    pltpu.sync_copy(ebuf, ep_hbm.at[pl.ds(wid * EPW, EPW)])

    for k in range(SUBC // 16):
        ones[pl.ds(k * 16, 16)] = jnp.full((16,), 1.0, jnp.float32)

    def zb(i, _):
        zbuf[pl.ds(i * 16, 16)] = jnp.zeros((16,), jnp.float32)
        return 0
    lax.fori_loop(0, CNT_SL // 16, zb, 0)
    pltpu.sync_copy(zbuf, acc.at[pl.ds(s * CNT_SL, CNT_SL)])
    plsc.subcore_barrier()

    def sget(j, si):
        base = j * SUBC
        for m in range(SUBC // 16):
            dv = dbuf[pl.ds(base + m * 16, 16)]
            tv = tbuf[pl.ds(base + m * 16, 16)]
            si[pl.ds(m * 16, 16)] = dv * NREL + tv

    sget(0, sidx)
    pltpu.async_copy(ones, acc.at[sidx], semb0, add=True)
    sget(1, sidx2)
    pltpu.async_copy(ones, acc.at[sidx2], semb1, add=True)

    def body(i, _):
        j0 = 2 * i
        pltpu.make_async_copy(ones, acc.at[sidx], semb0).wait()
        sget(j0 + 2, sidx)
        pltpu.async_copy(ones, acc.at[sidx], semb0, add=True)
        pltpu.make_async_copy(ones, acc.at[sidx2], semb1).wait()

        @pl.when(i < NFULL // 2 - 2)
        def _():
            sget(j0 + 3, sidx2)
            pltpu.async_copy(ones, acc.at[sidx2], semb1, add=True)
        return 0
    lax.fori_loop(0, NFULL // 2 - 1, body, 0)
    pltpu.make_async_copy(ones, acc.at[sidx], semb0).wait()
    sget(NFULL - 1, sidx2)
    pltpu.sync_copy(ones, acc.at[sidx2], add=True)
    dv = dbuf[pl.ds(NFULL * SUBC, TAIL)]
    tv = tbuf[pl.ds(NFULL * SUBC, TAIL)]
    sidxt[...] = dv * NREL + tv
    pltpu.sync_copy(ones.at[pl.ds(0, TAIL)], acc.at[sidxt], add=True)

    plsc.subcore_barrier()
    pltpu.sync_copy(acc.at[pl.ds(s * CNT_SL, CNT_SL)],
                    cnt_hbm.at[c, pl.ds(s * CNT_SL, CNT_SL)])


_cnt_call = functools.partial(
    pl.kernel,
    out_type=[jax.ShapeDtypeStruct((NC, NPAD), jnp.float32),
              jax.ShapeDtypeStruct((E,), jnp.int32)],
    mesh=_mesh,
    compiler_params=pltpu.CompilerParams(needs_layout_passes=False),
    scratch_types=[
        pltpu.VMEM((EPW,), jnp.int32),
        pltpu.VMEM((EPW,), jnp.int32),
        pltpu.VMEM((EPW,), jnp.int32),
        pltpu.VMEM((EPW,), jnp.int32),
        pltpu.VMEM((SUBC,), jnp.int32),
        pltpu.VMEM((SUBC,), jnp.int32),
        pltpu.VMEM((TAIL,), jnp.int32),
        pltpu.VMEM((SUBC,), jnp.float32),
        pltpu.VMEM((CNT_SL,), jnp.float32),
        pltpu.SemaphoreType.DMA,
        pltpu.SemaphoreType.DMA,
        pltpu.VMEM_SHARED((NPAD,), jnp.float32),
    ],
)(_cnt_body)


def _inv_body(cnt_ref, inv_ref):
    cb = cnt_ref[...]
    inv_ref[...] = 1.0 / jnp.maximum(cb[0] + cb[1], 1.0)


def _inv_call(cnt3):
    nr = NPAD // D
    return pl.pallas_call(
        _inv_body,
        grid=(1,),
        in_specs=[pl.BlockSpec((NC, nr, D), lambda i: (0, 0, 0))],
        out_specs=pl.BlockSpec((nr, D), lambda i: (0, 0)),
        out_shape=jax.ShapeDtypeStruct((nr, D), jnp.float32),
    )(cnt3)


NITER = NFULL // 2


def _main_body(y_hbm, ep_hbm, inv_hbm, rb_hbm, out0_hbm, out1_hbm,
               pbuf, rows0, rows1, scale0, scale1,
               gidx0, didx0, sidx0, gidx1, didx1, sidx1,
               gidxt, didxt, sidxt,
               semg0, semg1, sems0, sems1, acc):
    c = lax.axis_index("c")
    s = lax.axis_index("s")
    wid = _wid()

    pltpu.sync_copy(ep_hbm.at[pl.ds(wid * EPW, EPW)], pbuf)

    z0 = s * ZROWS
    rem = ZROWS - (ZROWS // SUBC) * SUBC

    @pl.when(c == 0)
    def _():
        for q in range(ZROWS // SUBC):
            pltpu.sync_copy(rb_hbm.at[pl.ds(z0 + q * SUBC, SUBC)],
                            acc.at[pl.ds(z0 + q * SUBC, SUBC)])
        pltpu.sync_copy(rb_hbm.at[pl.ds(z0 + (ZROWS // SUBC) * SUBC, rem)],
                        acc.at[pl.ds(z0 + (ZROWS // SUBC) * SUBC, rem)])

        @pl.when(s == NS - 1)
        def _():
            pltpu.sync_copy(rb_hbm.at[pl.ds(NS * ZROWS, N - NS * ZROWS)],
                            acc.at[pl.ds(NS * ZROWS, N - NS * ZROWS)])

    @pl.when(c == 1)
    def _():
        def zrow(r, _):
            for cc in range(D // 16):
                rows0[r, pl.ds(cc * 16, 16)] = jnp.zeros((16,), jnp.float32)
            return 0
        lax.fori_loop(0, SUBC, zrow, 0)
        for q in range(ZROWS // SUBC):
            pltpu.sync_copy(rows0, acc.at[pl.ds(z0 + q * SUBC, SUBC)])
        pltpu.sync_copy(rows0.at[pl.ds(0, rem)],
                        acc.at[pl.ds(z0 + (ZROWS // SUBC) * SUBC, rem)])

        @pl.when(s == NS - 1)
        def _():
            pltpu.sync_copy(rows0.at[pl.ds(0, N - NS * ZROWS)],
                            acc.at[pl.ds(NS * ZROWS, N - NS * ZROWS)])
    plsc.subcore_barrier()

    def prep(j, gi, di, si, rw, sc, sg):
        base = j * SUBC
        for m in range(SUBC // 16):
            p = pbuf[pl.ds(base + m * 16, 16)]
            sv, dv, tv = _unpack(p)
            gi[pl.ds(m * 16, 16)] = tv * N + sv
            di[pl.ds(m * 16, 16)] = dv
            si[pl.ds(m * 16, 16)] = dv * NREL + tv
        pltpu.async_copy(y_hbm.at[gi], rw, sg)
        pltpu.async_copy(inv_hbm.at[si], sc, sg)

    def proc(gi, di, si, rw, sc, sg, ss):
        pltpu.make_async_copy(y_hbm.at[gi], rw, sg).wait()
        pltpu.make_async_copy(inv_hbm.at[si], sc, sg).wait()

        def mrow(r, _):
            s16 = plsc.load_gather(sc, [jnp.full((16,), r, jnp.int32)])
            for cc in range(D // 16):
                rw[r, pl.ds(cc * 16, 16)] = rw[r, pl.ds(cc * 16, 16)] * s16
            return 0
        lax.fori_loop(0, SUBC, mrow, 0, unroll=8)
        pltpu.async_copy(rw, acc.at[di], ss, add=True)

    def drain(rw, di, ss):
        pltpu.make_async_copy(rw, acc.at[di], ss).wait()

    set0 = (gidx0, didx0, sidx0, rows0, scale0, semg0)
    set1 = (gidx1, didx1, sidx1, rows1, scale1, semg1)

    prep(0, *set0)
    prep(1, *set1)

    def body(i, _):
        j0 = 2 * i
        proc(*set0[:5], semg0, sems0)
        proc(*set1[:5], semg1, sems1)

        @pl.when(i < NITER - 1)
        def _():
            drain(rows0, didx0, sems0)
            prep(j0 + 2, *set0)
            drain(rows1, didx1, sems1)
            prep(j0 + 3, *set1)
        return 0
    lax.fori_loop(0, NITER, body, 0)
    drain(rows0, didx0, sems0)
    drain(rows1, didx1, sems1)

    p = pbuf[pl.ds(NFULL * SUBC, TAIL)]
    sv, dv, tv = _unpack(p)
    gidxt[...] = tv * N + sv
    didxt[...] = dv
    sidxt[...] = dv * NREL + tv
    cp1 = pltpu.async_copy(y_hbm.at[gidxt], rows0.at[pl.ds(0, TAIL)], semg0)
    cp2 = pltpu.async_copy(inv_hbm.at[sidxt], scale0.at[pl.ds(0, TAIL)], semg0)
    cp1.wait()
    cp2.wait()

    def mrowt(r, _):
        s16 = plsc.load_gather(scale0, [jnp.full((16,), r, jnp.int32)])
        for cc in range(D // 16):
            rows0[r, pl.ds(cc * 16, 16)] = rows0[r, pl.ds(cc * 16, 16)] * s16
        return 0
    lax.fori_loop(0, TAIL, mrowt, 0)
    pltpu.sync_copy(rows0.at[pl.ds(0, TAIL)], acc.at[didxt], add=True)

    plsc.subcore_barrier()
    r0 = s * ZROWS
    for cc_, o_hbm in ((0, out0_hbm), (1, out1_hbm)):
        @pl.when(c == cc_)
        def _(o_hbm=o_hbm):
            for q in range(ZROWS // SUBC):
                pltpu.sync_copy(acc.at[pl.ds(r0 + q * SUBC, SUBC)],
                                o_hbm.at[pl.ds(r0 + q * SUBC, SUBC)])
            pltpu.sync_copy(acc.at[pl.ds(r0 + (ZROWS // SUBC) * SUBC, rem)],
                            o_hbm.at[pl.ds(r0 + (ZROWS // SUBC) * SUBC, rem)])

            @pl.when(s == NS - 1)
            def _():
                pltpu.sync_copy(acc.at[pl.ds(NS * ZROWS, N - NS * ZROWS)],
                                o_hbm.at[pl.ds(NS * ZROWS, N - NS * ZROWS)])


_main_call = functools.partial(
    pl.kernel,
    out_type=[jax.ShapeDtypeStruct((N, D), jnp.float32),
              jax.ShapeDtypeStruct((N, D), jnp.float32)],
    mesh=_mesh,
    compiler_params=pltpu.CompilerParams(needs_layout_passes=False),
    scratch_types=[
        pltpu.VMEM((EPW,), jnp.int32),
        pltpu.VMEM((SUBC, D), jnp.float32),
        pltpu.VMEM((SUBC, D), jnp.float32),
        pltpu.VMEM((SUBC,), jnp.float32),
        pltpu.VMEM((SUBC,), jnp.float32),
        pltpu.VMEM((SUBC,), jnp.int32),
        pltpu.VMEM((SUBC,), jnp.int32),
        pltpu.VMEM((SUBC,), jnp.int32),
        pltpu.VMEM((SUBC,), jnp.int32),
        pltpu.VMEM((SUBC,), jnp.int32),
        pltpu.VMEM((SUBC,), jnp.int32),
        pltpu.VMEM((TAIL,), jnp.int32),
        pltpu.VMEM((TAIL,), jnp.int32),
        pltpu.VMEM((TAIL,), jnp.int32),
        pltpu.SemaphoreType.DMA,
        pltpu.SemaphoreType.DMA,
        pltpu.SemaphoreType.DMA,
        pltpu.SemaphoreType.DMA,
        pltpu.VMEM_SHARED((N, D), jnp.float32),
    ],
)(_main_body)


_EPT = B // NW


def _emb_body(m0_hbm, m1_hbm, nest_hbm, food_hbm,
              ne_hbm, fe_hbm, idxn, idxf, m0n, m1n, m0f, m1f, semn, semf):
    wid = _wid()
    base = wid * _EPT
    pltpu.sync_copy(nest_hbm.at[pl.ds(base, _EPT)], idxn)
    pltpu.sync_copy(food_hbm.at[pl.ds(base, _EPT)], idxf)
    pltpu.async_copy(m0_hbm.at[idxn], m0n, semn)
    pltpu.async_copy(m1_hbm.at[idxn], m1n, semn)
    pltpu.async_copy(m0_hbm.at[idxf], m0f, semf)
    pltpu.async_copy(m1_hbm.at[idxf], m1f, semf)

    def combine(m0b, m1b):
        def row(r, _):
            for cc in range(D // 16):
                ds = pl.ds(cc * 16, 16)
                m0b[r, ds] = jnp.maximum(m0b[r, ds] + m1b[r, ds], 0.0)
            return 0
        lax.fori_loop(0, _EPT, row, 0, unroll=4)

    pltpu.make_async_copy(m0_hbm.at[idxn], m0n, semn).wait()
    pltpu.make_async_copy(m1_hbm.at[idxn], m1n, semn).wait()
    combine(m0n, m1n)
    pltpu.async_copy(m0n, ne_hbm.at[pl.ds(base, _EPT)], semn)

    pltpu.make_async_copy(m0_hbm.at[idxf], m0f, semf).wait()
    pltpu.make_async_copy(m1_hbm.at[idxf], m1f, semf).wait()
    combine(m0f, m1f)
    pltpu.async_copy(m0f, fe_hbm.at[pl.ds(base, _EPT)], semf)

    pltpu.make_async_copy(m0n, ne_hbm.at[pl.ds(base, _EPT)], semn).wait()
    pltpu.make_async_copy(m0f, fe_hbm.at[pl.ds(base, _EPT)], semf).wait()


_emb_call = functools.partial(
    pl.kernel,
    out_type=[jax.ShapeDtypeStruct((B, D), jnp.float32),
              jax.ShapeDtypeStruct((B, D), jnp.float32)],
    mesh=_mesh,
    compiler_params=pltpu.CompilerParams(needs_layout_passes=False),
    scratch_types=[
        pltpu.VMEM((_EPT,), jnp.int32),
        pltpu.VMEM((_EPT,), jnp.int32),
        pltpu.VMEM((_EPT, D), jnp.float32),
        pltpu.VMEM((_EPT, D), jnp.float32),
        pltpu.VMEM((_EPT, D), jnp.float32),
        pltpu.VMEM((_EPT, D), jnp.float32),
        pltpu.SemaphoreType.DMA,
        pltpu.SemaphoreType.DMA,
    ],
)(_emb_body)


def _head_body(ne_ref, fe_ref, wfc_ref, bfc_ref, wdir_ref, bdir_ref,
               wdist_ref, bdist_ref, la_ref, tb_ref):
    hid = jnp.dot(ne_ref[...], wfc_ref[:D], preferred_element_type=jnp.float32)
    hid = hid + jnp.dot(fe_ref[...], wfc_ref[D:],
                        preferred_element_type=jnp.float32)
    hid = jnp.maximum(hid + bfc_ref[...], 0.0)
    logit = jnp.dot(hid, wdir_ref[...], preferred_element_type=jnp.float32)
    logit = logit + bdir_ref[...]
    m = jnp.max(logit, axis=-1, keepdims=True)
    lse = jnp.log(jnp.sum(jnp.exp(logit - m), axis=-1, keepdims=True)) + m
    la_ref[...] = logit - lse
    tb_ref[...] = (jnp.dot(hid, wdist_ref[...],
                           preferred_element_type=jnp.float32)
                   + bdist_ref[...])


def _head_call(ne, fe, W_fc, b_fc, W_dir, b_dir, W_dist, b_dist):
    bm = 2048
    grid = (B // bm,)
    return pl.pallas_call(
        _head_body,
        grid=grid,
        in_specs=[
            pl.BlockSpec((bm, D), lambda i: (i, 0)),
            pl.BlockSpec((bm, D), lambda i: (i, 0)),
            pl.BlockSpec((2 * D, HIDDEN), lambda i: (0, 0)),
            pl.BlockSpec((1, HIDDEN), lambda i: (0, 0)),
            pl.BlockSpec((HIDDEN, VOCAB), lambda i: (0, 0)),
            pl.BlockSpec((1, VOCAB), lambda i: (0, 0)),
            pl.BlockSpec((HIDDEN, 1), lambda i: (0, 0)),
            pl.BlockSpec((1, 1), lambda i: (0, 0)),
        ],
        out_specs=[
            pl.BlockSpec((bm, VOCAB), lambda i: (i, 0)),
            pl.BlockSpec((bm, 1), lambda i: (i, 0)),
        ],
        out_shape=[
            jax.ShapeDtypeStruct((B, VOCAB), jnp.float32),
            jax.ShapeDtypeStruct((B, 1), jnp.float32),
        ],
    )(ne, fe, W_fc, b_fc, W_dir, b_dir, W_dist, b_dist)


def kernel(x, edge_index, edge_type, nest, food, W_rel, W_root, b_rgcn,
           W_fc, b_fc, W_dir, b_dir, W_dist, b_dist):
    src = edge_index[0].astype(jnp.int32)
    dst = edge_index[1].astype(jnp.int32)
    et = edge_type.astype(jnp.int32)
    nest32 = nest.astype(jnp.int32)
    food32 = food.astype(jnp.int32)

    Y, rootb = _mm_call(x, W_rel, W_root, b_rgcn.reshape(1, -1))
    Y2 = Y.reshape(NREL * N, D)
    cnt2, epack = _cnt_call(src, dst, et)
    inv = _inv_call(cnt2.reshape(NC, NPAD // D, D)).reshape(NPAD)
    msg0, msg1 = _main_call(Y2, epack, inv, rootb)
    ne, fe = _emb_call(msg0, msg1, nest32, food32)
    la, tb = _head_call(ne, fe, W_fc, b_fc.reshape(1, -1),
                        W_dir, b_dir.reshape(1, -1),
                        W_dist, b_dist.reshape(1, -1))
    return (la, tb)
